# baseline (speedup 1.0000x reference)
import functools

import jax
import jax.numpy as jnp
from jax import lax
from jax.experimental import pallas as pl
from jax.experimental.pallas import tpu as pltpu
from jax.experimental.pallas import tpu_sc as plsc

N = 10000
E = 320000
D = 128
R = 4
G = 8
NC = 2
NS = 16
NW = NC * NS

E_PAD = 327680
EPT = E_PAD // NW
EPC = E_PAD // NS
CH = 128
WCH = 2048
RNP = 40960
SL = RNP // NS
N_PAD = 10240
STRIPE = N_PAD // NS
BN = 400
NB = N // BN

_sc_mesh = plsc.VectorSubcoreMesh(
    core_axis_name="c", subcore_axis_name="s", num_cores=NC, num_subcores=NS)


@functools.partial(
    pl.kernel,
    out_type=[jax.ShapeDtypeStruct((E_PAD,), jnp.int32),
              jax.ShapeDtypeStruct((E_PAD,), jnp.float32)],
    mesh=_sc_mesh,
    scratch_types=[
        pltpu.VMEM_SHARED((RNP,), jnp.float32),
        pltpu.VMEM((RNP,), jnp.float32),
        pltpu.VMEM((SL,), jnp.float32),
        pltpu.VMEM((CH,), jnp.int32),
        pltpu.VMEM((CH,), jnp.int32),
        pltpu.VMEM((CH,), jnp.int32),
        pltpu.VMEM((CH,), jnp.float32),
        pltpu.VMEM((WCH,), jnp.int32),
        pltpu.VMEM((WCH,), jnp.int32),
        pltpu.VMEM((WCH,), jnp.int32),
        pltpu.VMEM((WCH,), jnp.int32),
        pltpu.VMEM((WCH,), jnp.float32),
    ],
    compiler_params=pltpu.CompilerParams(needs_layout_passes=False),
)
def _count_weights(et_hbm, src_hbm, dst_hbm, g_hbm, w_hbm,
                   c_sh, cl, zb, etb, dstb, sidxb, valb,
                   etb2, srcb2, dstb2, gb, wb):
    cid = lax.axis_index("c")
    sid = lax.axis_index("s")
    zero16 = jnp.zeros((16,), jnp.float32)
    iota16 = lax.broadcasted_iota(jnp.int32, (16,), 0)

    def _z(i, carry):
        zb[pl.ds(i * 16, 16)] = zero16
        return carry
    lax.fori_loop(0, SL // 16, _z, 0)
    pltpu.sync_copy(zb, c_sh.at[pl.ds(sid * SL, SL)])
    plsc.subcore_barrier()

    def _cchunk(j, carry):
        base = sid * EPC + j * CH
        pltpu.sync_copy(et_hbm.at[pl.ds(base, CH)], etb)
        pltpu.sync_copy(dst_hbm.at[pl.ds(base, CH)], dstb)

        def _grp(k, c2):
            o = k * 16
            s16 = etb[pl.ds(o, 16)] * N + dstb[pl.ds(o, 16)]
            v16 = jnp.where(base + o + iota16 < E, 1.0, 0.0)
            sidxb[pl.ds(o, 16)] = s16
            valb[pl.ds(o, 16)] = v16
            return c2
        lax.fori_loop(0, CH // 16, _grp, 0)
        pltpu.sync_copy(valb, c_sh.at[sidxb], add=True)
        return carry
    lax.fori_loop(0, EPC // CH, _cchunk, 0)
    plsc.subcore_barrier()
    pltpu.sync_copy(c_sh, cl)

    wid = cid * NS + sid

    def _wchunk(j, carry):
        base = wid * EPT + j * WCH
        pltpu.sync_copy(et_hbm.at[pl.ds(base, WCH)], etb2)
        pltpu.sync_copy(src_hbm.at[pl.ds(base, WCH)], srcb2)
        pltpu.sync_copy(dst_hbm.at[pl.ds(base, WCH)], dstb2)

        def _grp(k, c2):
            o = k * 16
            et16 = etb2[pl.ds(o, 16)]
            gb[pl.ds(o, 16)] = et16 * N + srcb2[pl.ds(o, 16)]
            s16 = et16 * N + dstb2[pl.ds(o, 16)]
            c16 = plsc.load_gather(cl, [s16])
            w16 = jnp.where(base + o + iota16 < E,
                            1.0 / jnp.maximum(c16, 1.0), 0.0)
            wb[pl.ds(o, 16)] = w16
            return c2
        lax.fori_loop(0, WCH // 16, _grp, 0)
        pltpu.sync_copy(gb, g_hbm.at[pl.ds(base, WCH)])
        pltpu.sync_copy(wb, w_hbm.at[pl.ds(base, WCH)])
        return carry
    lax.fori_loop(0, EPT // WCH, _wchunk, 0)


NCHT = EPT // CH
_DBITS = 14
NCH0 = 40
NCH1 = 120


@functools.partial(
    pl.kernel,
    out_type=jax.ShapeDtypeStruct((NC, N_PAD, D), jnp.float32),
    mesh=_sc_mesh,
    scratch_types=[
        pltpu.VMEM_SHARED((N_PAD, D), jnp.float32),
        pltpu.VMEM((CH,), jnp.int32),
        pltpu.VMEM((CH,), jnp.int32),
        pltpu.VMEM((CH // 2,), jnp.int32),
        pltpu.VMEM((CH // 2,), jnp.int32),
        pltpu.VMEM((CH,), jnp.int32),
        pltpu.VMEM((CH,), jnp.int32),
        pltpu.VMEM((CH,), jnp.int32),
        pltpu.VMEM((CH,), jnp.int32),
        pltpu.VMEM((CH, D), jnp.float32),
        pltpu.VMEM((CH, D), jnp.float32),
        pltpu.SemaphoreType.DMA,
        pltpu.SemaphoreType.DMA,
        pltpu.SemaphoreType.DMA,
        pltpu.SemaphoreType.DMA,
        pltpu.SemaphoreType.DMA,
        pltpu.SemaphoreType.DMA,
    ],
    compiler_params=pltpu.CompilerParams(needs_layout_passes=False),
)
def _scatter(y_hbm, pk_hbm, w_hbm, p_hbm,
             a_sh, pkr0, pkr1, wr0, wr1, gi0, gi1, di0, di1, rows0, rows1,
             sp0, sp1, sg0, sg1, ss0, ss1):
    cid = lax.axis_index("c")
    sid = lax.axis_index("s")
    zero16 = jnp.zeros((16,), jnp.float32)

    def _z(i, carry):
        rows0[i // 8, pl.ds((i % 8) * 16, 16)] = zero16
        return carry
    lax.fori_loop(0, (CH * D) // 16, _z, 0)

    def _zs(t, carry):
        pltpu.sync_copy(rows0, a_sh.at[pl.ds(sid * STRIPE + t * CH, CH)])
        return carry
    lax.fori_loop(0, STRIPE // CH, _zs, 0)
    plsc.subcore_barrier()

    ncht = jnp.where(cid == 0, NCH0, NCH1)
    cb = jnp.where(cid == 0, sid * NCH0, NS * NCH0 + sid * NCH1)

    def _unpack(pkr_s, gi_s, di_s):
        for k in range(CH // 16):
            o = k * 16
            p16 = pkr_s[pl.ds(o, 16)]
            gi_s[pl.ds(o, 16)] = lax.shift_right_logical(p16, _DBITS)
            di_s[pl.ds(o, 16)] = lax.bitwise_and(p16, (1 << _DBITS) - 1)

    bufs = ((rows0, pkr0, wr0, gi0, di0, sp0, sg0, ss0),
            (rows1, pkr1, wr1, gi1, di1, sp1, sg1, ss1))

    pltpu.sync_copy(pk_hbm.at[cb], pkr0)
    pltpu.sync_copy(w_hbm.at[cb], wr0)
    _unpack(pkr0, gi0, di0)
    pltpu.async_copy(y_hbm.at[gi0], rows0, sg0)
    pltpu.async_copy(pk_hbm.at[cb + 1], pkr1, sp1)
    pltpu.async_copy(w_hbm.at[cb + 1], wr1, sp1)

    def _pair(jj, carry):
        for b in range(2):
            j = jj * 2 + b
            rows_b, pkr_b, wr_b, gi_b, di_b, sp_b, sg_b, ss_b = bufs[b]
            rows_n, pkr_n, wr_n, gi_n, di_n, sp_n, sg_n, ss_n = bufs[1 - b]

            @pl.when(j + 1 < ncht)
            def _():
                @pl.when(j >= 1)
                def _():
                    pltpu.make_async_copy(rows_n, a_sh.at[di_n], ss_n).wait()
                pltpu.make_async_copy(pk_hbm.at[cb], pkr_n, sp_n).wait()
                pltpu.make_async_copy(w_hbm.at[cb], wr_n, sp_n).wait()
                _unpack(pkr_n, gi_n, di_n)
                pltpu.async_copy(y_hbm.at[gi_n], rows_n, sg_n)

            pltpu.make_async_copy(y_hbm.at[gi_b], rows_b, sg_b).wait()

            def _scale(k, c2):
                wp16 = wr_b[pl.ds(k * 16, 16)]
                we = plsc.bitcast(lax.shift_left(wp16, 16), jnp.float32)
                wo = plsc.bitcast(lax.bitwise_and(wp16, jnp.int32(-65536)),
                                  jnp.float32)
                for l in range(16):
                    e0 = k * 32 + 2 * l
                    wl0 = jnp.full((16,), we[l], jnp.float32)
                    wl1 = jnp.full((16,), wo[l], jnp.float32)
                    for c in range(D // 16):
                        rows_b[e0, pl.ds(c * 16, 16)] = (
                            rows_b[e0, pl.ds(c * 16, 16)] * wl0)
                        rows_b[e0 + 1, pl.ds(c * 16, 16)] = (
                            rows_b[e0 + 1, pl.ds(c * 16, 16)] * wl1)
                return c2
            lax.fori_loop(0, CH // 32, _scale, 0)
            pltpu.async_copy(rows_b, a_sh.at[di_b], ss_b, add=True)

            @pl.when(j + 2 < ncht)
            def _():
                pltpu.async_copy(pk_hbm.at[cb + j + 2], pkr_b, sp_b)
                pltpu.async_copy(w_hbm.at[cb + j + 2], wr_b, sp_b)
        return carry
    lax.fori_loop(0, lax.div(ncht, 2), _pair, 0)
    pltpu.make_async_copy(rows0, a_sh.at[di0], ss0).wait()
    pltpu.make_async_copy(rows1, a_sh.at[di1], ss1).wait()
    plsc.subcore_barrier()

    def _out(t, carry):
        off = sid * STRIPE + t * CH
        pltpu.sync_copy(a_sh.at[pl.ds(off, CH)], p_hbm.at[cid, pl.ds(off, CH)])
        return carry
    lax.fori_loop(0, STRIPE // CH, _out, 0)


def _mm1_body(x_ref, root_ref, w_ref, b_ref, base_ref, y_ref):
    xb = x_ref[...]
    base_ref[...] = jnp.dot(xb, root_ref[...],
                            preferred_element_type=jnp.float32) + b_ref[...]
    for r in range(R):
        y_ref[r] = jnp.dot(xb, w_ref[r], preferred_element_type=jnp.float32)


def _mm1(x, root, w, b):
    return pl.pallas_call(
        _mm1_body,
        grid=(NB,),
        in_specs=[pl.BlockSpec((BN, D), lambda i: (i, 0)),
                  pl.BlockSpec((D, D), lambda i: (0, 0)),
                  pl.BlockSpec((R, D, D), lambda i: (0, 0, 0)),
                  pl.BlockSpec((1, D), lambda i: (0, 0))],
        out_specs=[pl.BlockSpec((BN, D), lambda i: (i, 0)),
                   pl.BlockSpec((R, BN, D), lambda i: (0, i, 0))],
        out_shape=[jax.ShapeDtypeStruct((N, D), jnp.float32),
                   jax.ShapeDtypeStruct((R, N, D), jnp.float32)],
    )(x, root, w, b)


def _mm2_body(base_ref, p_ref, root_ref, w_ref, b_ref, base2_ref, y_ref):
    h = jnp.maximum(base_ref[...] + p_ref[0].astype(jnp.float32)
                    + p_ref[1].astype(jnp.float32), 0.0)
    base2_ref[...] = jnp.dot(h, root_ref[...],
                             preferred_element_type=jnp.float32) + b_ref[...]
    for r in range(R):
        y_ref[r] = jnp.dot(h, w_ref[r], preferred_element_type=jnp.float32)


def _mm2(base, p, root, w, b):
    return pl.pallas_call(
        _mm2_body,
        grid=(NB,),
        in_specs=[pl.BlockSpec((BN, D), lambda i: (i, 0)),
                  pl.BlockSpec((NC, BN, D), lambda i: (0, i, 0)),
                  pl.BlockSpec((D, D), lambda i: (0, 0)),
                  pl.BlockSpec((R, D, D), lambda i: (0, 0, 0)),
                  pl.BlockSpec((1, D), lambda i: (0, 0))],
        out_specs=[pl.BlockSpec((BN, D), lambda i: (i, 0)),
                   pl.BlockSpec((R, BN, D), lambda i: (0, i, 0))],
        out_shape=[jax.ShapeDtypeStruct((N, D), jnp.float32),
                   jax.ShapeDtypeStruct((R, N, D), jnp.float32)],
    )(base, p, root, w, b)


def _pool_body(base_ref, p_ref, batch_ref, linw_ref, linb_ref, out_ref,
               sums, cnts):
    i = pl.program_id(0)

    @pl.when(i == 0)
    def _():
        sums[...] = jnp.zeros((G, D), jnp.float32)
        cnts[...] = jnp.zeros((G, D), jnp.float32)

    h = jnp.maximum(base_ref[...] + p_ref[0].astype(jnp.float32)
                    + p_ref[1].astype(jnp.float32), 0.0)
    b = batch_ref[...]
    oh = (b == lax.broadcasted_iota(jnp.int32, (BN, G), 1)).astype(jnp.float32)
    sums[...] += lax.dot_general(oh, h, (((0,), (0,)), ((), ())),
                                 preferred_element_type=jnp.float32)
    cnts[...] += jnp.sum(oh, axis=0)[:, None]

    @pl.when(i == NB - 1)
    def _():
        pooled = sums[...] / jnp.maximum(cnts[...], 1.0)
        out_ref[...] = jnp.dot(pooled, linw_ref[...],
                               preferred_element_type=jnp.float32) + linb_ref[...]


def _pool(base, p, batch, linw, linb):
    return pl.pallas_call(
        _pool_body,
        grid=(NB,),
        in_specs=[pl.BlockSpec((BN, D), lambda i: (i, 0)),
                  pl.BlockSpec((NC, BN, D), lambda i: (0, i, 0)),
                  pl.BlockSpec((BN, 1), lambda i: (i, 0)),
                  pl.BlockSpec((D, D), lambda i: (0, 0)),
                  pl.BlockSpec((1, D), lambda i: (0, 0))],
        out_specs=pl.BlockSpec((G, D), lambda i: (0, 0)),
        out_shape=jax.ShapeDtypeStruct((G, D), jnp.float32),
        scratch_shapes=[pltpu.VMEM((G, D), jnp.float32),
                        pltpu.VMEM((G, D), jnp.float32)],
    )(base, p, batch, linw, linb)


def kernel(x, edge_index, edge_type, batch, W1, root1, b1, W2, root2, b2,
           linW, linb):
    src = edge_index[0].astype(jnp.int32)
    dst = edge_index[1].astype(jnp.int32)
    et = edge_type.astype(jnp.int32)
    pad = E_PAD - E
    src_p = jnp.pad(src, (0, pad))
    dst_p = jnp.pad(dst, (0, pad))
    et_p = jnp.pad(et, (0, pad))

    g_idx, w_edge = _count_weights(et_p, src_p, dst_p)
    pk2 = ((g_idx << _DBITS) | dst_p).reshape(E_PAD // CH, CH)
    wpk2 = jax.lax.bitcast_convert_type(
        w_edge.astype(jnp.bfloat16).reshape(E_PAD // 2, 2), jnp.int32
    ).reshape(E_PAD // CH, CH // 2)

    base1, y1 = _mm1(x, root1, W1, b1.reshape(1, D))
    p1 = _scatter(y1.reshape(R * N, D), pk2, wpk2)
    base2, y2 = _mm2(base1, p1, root2, W2, b2.reshape(1, D))
    p2 = _scatter(y2.reshape(R * N, D), pk2, wpk2)

    linWp = jnp.zeros((D, D), jnp.float32).at[:, :2].set(linW)
    linbp = jnp.zeros((1, D), jnp.float32).at[0, :2].set(linb)
    out = _pool(base2, p2, batch.astype(jnp.int32).reshape(N, 1),
                linWp, linbp)
    return out[:, :2]


# speedup vs baseline: 1.1153x; 1.1153x over previous
import functools

import jax
import jax.numpy as jnp
from jax import lax
from jax.experimental import pallas as pl
from jax.experimental.pallas import tpu as pltpu
from jax.experimental.pallas import tpu_sc as plsc

N = 10000
E = 320000
D = 128
R = 4
G = 8
NC = 2
NS = 16
NW = NC * NS

E_PAD = 327680
EPT = E_PAD // NW
EPC = E_PAD // NS
CH = 128
WCH = 2048
RNP = 40960
SL = RNP // NS
N_PAD = 10240
STRIPE = N_PAD // NS
BN = 400
NB = N // BN

_sc_mesh = plsc.VectorSubcoreMesh(
    core_axis_name="c", subcore_axis_name="s", num_cores=NC, num_subcores=NS)


@functools.partial(
    pl.kernel,
    out_type=[jax.ShapeDtypeStruct((E_PAD,), jnp.int32),
              jax.ShapeDtypeStruct((E_PAD,), jnp.float32)],
    mesh=_sc_mesh,
    scratch_types=[
        pltpu.VMEM_SHARED((RNP,), jnp.float32),
        pltpu.VMEM((RNP,), jnp.float32),
        pltpu.VMEM((SL,), jnp.float32),
        pltpu.VMEM((CH,), jnp.int32),
        pltpu.VMEM((CH,), jnp.int32),
        pltpu.VMEM((CH,), jnp.int32),
        pltpu.VMEM((CH,), jnp.float32),
        pltpu.VMEM((WCH,), jnp.int32),
        pltpu.VMEM((WCH,), jnp.int32),
        pltpu.VMEM((WCH,), jnp.int32),
        pltpu.VMEM((WCH,), jnp.int32),
        pltpu.VMEM((WCH,), jnp.float32),
    ],
    compiler_params=pltpu.CompilerParams(needs_layout_passes=False),
)
def _count_weights(et_hbm, src_hbm, dst_hbm, g_hbm, w_hbm,
                   c_sh, cl, zb, etb, dstb, sidxb, valb,
                   etb2, srcb2, dstb2, gb, wb):
    cid = lax.axis_index("c")
    sid = lax.axis_index("s")
    zero16 = jnp.zeros((16,), jnp.float32)
    iota16 = lax.broadcasted_iota(jnp.int32, (16,), 0)

    def _z(i, carry):
        zb[pl.ds(i * 16, 16)] = zero16
        return carry
    lax.fori_loop(0, SL // 16, _z, 0)
    pltpu.sync_copy(zb, c_sh.at[pl.ds(sid * SL, SL)])
    plsc.subcore_barrier()

    def _cchunk(j, carry):
        base = sid * EPC + j * CH
        pltpu.sync_copy(et_hbm.at[pl.ds(base, CH)], etb)
        pltpu.sync_copy(dst_hbm.at[pl.ds(base, CH)], dstb)

        def _grp(k, c2):
            o = k * 16
            s16 = etb[pl.ds(o, 16)] * N + dstb[pl.ds(o, 16)]
            v16 = jnp.where(base + o + iota16 < E, 1.0, 0.0)
            sidxb[pl.ds(o, 16)] = s16
            valb[pl.ds(o, 16)] = v16
            return c2
        lax.fori_loop(0, CH // 16, _grp, 0)
        pltpu.sync_copy(valb, c_sh.at[sidxb], add=True)
        return carry
    lax.fori_loop(0, EPC // CH, _cchunk, 0)
    plsc.subcore_barrier()
    pltpu.sync_copy(c_sh, cl)

    wid = cid * NS + sid

    def _wchunk(j, carry):
        base = wid * EPT + j * WCH
        pltpu.sync_copy(et_hbm.at[pl.ds(base, WCH)], etb2)
        pltpu.sync_copy(src_hbm.at[pl.ds(base, WCH)], srcb2)
        pltpu.sync_copy(dst_hbm.at[pl.ds(base, WCH)], dstb2)

        def _grp(k, c2):
            o = k * 16
            et16 = etb2[pl.ds(o, 16)]
            gb[pl.ds(o, 16)] = et16 * N + srcb2[pl.ds(o, 16)]
            s16 = et16 * N + dstb2[pl.ds(o, 16)]
            c16 = plsc.load_gather(cl, [s16])
            w16 = jnp.where(base + o + iota16 < E,
                            1.0 / jnp.maximum(c16, 1.0), 0.0)
            wb[pl.ds(o, 16)] = w16
            return c2
        lax.fori_loop(0, WCH // 16, _grp, 0)
        pltpu.sync_copy(gb, g_hbm.at[pl.ds(base, WCH)])
        pltpu.sync_copy(wb, w_hbm.at[pl.ds(base, WCH)])
        return carry
    lax.fori_loop(0, EPT // WCH, _wchunk, 0)


NCHT = EPT // CH
_DBITS = 14
NCH0 = 120
NCH1 = 40


@functools.partial(
    pl.kernel,
    out_type=jax.ShapeDtypeStruct((NC, N_PAD, D), jnp.float32),
    mesh=_sc_mesh,
    scratch_types=[
        pltpu.VMEM_SHARED((N_PAD, D), jnp.float32),
        pltpu.VMEM((CH,), jnp.int32),
        pltpu.VMEM((CH,), jnp.int32),
        pltpu.VMEM((CH // 2,), jnp.int32),
        pltpu.VMEM((CH // 2,), jnp.int32),
        pltpu.VMEM((CH,), jnp.int32),
        pltpu.VMEM((CH,), jnp.int32),
        pltpu.VMEM((CH,), jnp.int32),
        pltpu.VMEM((CH,), jnp.int32),
        pltpu.VMEM((CH, D), jnp.float32),
        pltpu.VMEM((CH, D), jnp.float32),
        pltpu.SemaphoreType.DMA,
        pltpu.SemaphoreType.DMA,
        pltpu.SemaphoreType.DMA,
        pltpu.SemaphoreType.DMA,
        pltpu.SemaphoreType.DMA,
        pltpu.SemaphoreType.DMA,
    ],
    compiler_params=pltpu.CompilerParams(needs_layout_passes=False),
)
def _scatter(y_hbm, pk_hbm, w_hbm, p_hbm,
             a_sh, pkr0, pkr1, wr0, wr1, gi0, gi1, di0, di1, rows0, rows1,
             sp0, sp1, sg0, sg1, ss0, ss1):
    cid = lax.axis_index("c")
    sid = lax.axis_index("s")
    zero16 = jnp.zeros((16,), jnp.float32)

    def _z(i, carry):
        rows0[i // 8, pl.ds((i % 8) * 16, 16)] = zero16
        return carry
    lax.fori_loop(0, (CH * D) // 16, _z, 0)

    def _zs(t, carry):
        pltpu.sync_copy(rows0, a_sh.at[pl.ds(sid * STRIPE + t * CH, CH)])
        return carry
    lax.fori_loop(0, STRIPE // CH, _zs, 0)
    plsc.subcore_barrier()

    ncht = jnp.where(cid == 0, NCH0, NCH1)
    cb = jnp.where(cid == 0, sid * NCH0, NS * NCH0 + sid * NCH1)

    def _unpack(pkr_s, gi_s, di_s):
        for k in range(CH // 16):
            o = k * 16
            p16 = pkr_s[pl.ds(o, 16)]
            gi_s[pl.ds(o, 16)] = lax.shift_right_logical(p16, _DBITS)
            di_s[pl.ds(o, 16)] = lax.bitwise_and(p16, (1 << _DBITS) - 1)

    bufs = ((rows0, pkr0, wr0, gi0, di0, sp0, sg0, ss0),
            (rows1, pkr1, wr1, gi1, di1, sp1, sg1, ss1))

    pltpu.sync_copy(pk_hbm.at[cb], pkr0)
    pltpu.sync_copy(w_hbm.at[cb], wr0)
    _unpack(pkr0, gi0, di0)
    pltpu.async_copy(y_hbm.at[gi0], rows0, sg0)
    pltpu.async_copy(pk_hbm.at[cb + 1], pkr1, sp1)
    pltpu.async_copy(w_hbm.at[cb + 1], wr1, sp1)

    def _pair(jj, carry):
        for b in range(2):
            j = jj * 2 + b
            rows_b, pkr_b, wr_b, gi_b, di_b, sp_b, sg_b, ss_b = bufs[b]
            rows_n, pkr_n, wr_n, gi_n, di_n, sp_n, sg_n, ss_n = bufs[1 - b]

            @pl.when(j + 1 < ncht)
            def _():
                @pl.when(j >= 1)
                def _():
                    pltpu.make_async_copy(rows_n, a_sh.at[di_n], ss_n).wait()
                pltpu.make_async_copy(pk_hbm.at[cb], pkr_n, sp_n).wait()
                pltpu.make_async_copy(w_hbm.at[cb], wr_n, sp_n).wait()
                _unpack(pkr_n, gi_n, di_n)
                pltpu.async_copy(y_hbm.at[gi_n], rows_n, sg_n)

            pltpu.make_async_copy(y_hbm.at[gi_b], rows_b, sg_b).wait()

            def _scale(k, c2):
                wp16 = wr_b[pl.ds(k * 16, 16)]
                we = plsc.bitcast(lax.shift_left(wp16, 16), jnp.float32)
                wo = plsc.bitcast(lax.bitwise_and(wp16, jnp.int32(-65536)),
                                  jnp.float32)
                for l in range(16):
                    e0 = k * 32 + 2 * l
                    wl0 = jnp.full((16,), we[l], jnp.float32)
                    wl1 = jnp.full((16,), wo[l], jnp.float32)
                    for c in range(D // 16):
                        rows_b[e0, pl.ds(c * 16, 16)] = (
                            rows_b[e0, pl.ds(c * 16, 16)] * wl0)
                        rows_b[e0 + 1, pl.ds(c * 16, 16)] = (
                            rows_b[e0 + 1, pl.ds(c * 16, 16)] * wl1)
                return c2
            lax.fori_loop(0, CH // 32, _scale, 0)
            pltpu.async_copy(rows_b, a_sh.at[di_b], ss_b, add=True)

            @pl.when(j + 2 < ncht)
            def _():
                pltpu.async_copy(pk_hbm.at[cb + j + 2], pkr_b, sp_b)
                pltpu.async_copy(w_hbm.at[cb + j + 2], wr_b, sp_b)
        return carry
    lax.fori_loop(0, lax.div(ncht, 2), _pair, 0)
    pltpu.make_async_copy(rows0, a_sh.at[di0], ss0).wait()
    pltpu.make_async_copy(rows1, a_sh.at[di1], ss1).wait()
    plsc.subcore_barrier()

    def _out(t, carry):
        off = sid * STRIPE + t * CH
        pltpu.sync_copy(a_sh.at[pl.ds(off, CH)], p_hbm.at[cid, pl.ds(off, CH)])
        return carry
    lax.fori_loop(0, STRIPE // CH, _out, 0)


def _mm1_body(x_ref, root_ref, w_ref, b_ref, base_ref, y_ref):
    xb = x_ref[...]
    base_ref[...] = jnp.dot(xb, root_ref[...],
                            preferred_element_type=jnp.float32) + b_ref[...]
    for r in range(R):
        y_ref[r] = jnp.dot(xb, w_ref[r], preferred_element_type=jnp.float32)


def _mm1(x, root, w, b):
    return pl.pallas_call(
        _mm1_body,
        grid=(NB,),
        in_specs=[pl.BlockSpec((BN, D), lambda i: (i, 0)),
                  pl.BlockSpec((D, D), lambda i: (0, 0)),
                  pl.BlockSpec((R, D, D), lambda i: (0, 0, 0)),
                  pl.BlockSpec((1, D), lambda i: (0, 0))],
        out_specs=[pl.BlockSpec((BN, D), lambda i: (i, 0)),
                   pl.BlockSpec((R, BN, D), lambda i: (0, i, 0))],
        out_shape=[jax.ShapeDtypeStruct((N, D), jnp.float32),
                   jax.ShapeDtypeStruct((R, N, D), jnp.float32)],
    )(x, root, w, b)


def _mm2_body(base_ref, p_ref, root_ref, w_ref, b_ref, base2_ref, y_ref):
    h = jnp.maximum(base_ref[...] + p_ref[0].astype(jnp.float32)
                    + p_ref[1].astype(jnp.float32), 0.0)
    base2_ref[...] = jnp.dot(h, root_ref[...],
                             preferred_element_type=jnp.float32) + b_ref[...]
    for r in range(R):
        y_ref[r] = jnp.dot(h, w_ref[r], preferred_element_type=jnp.float32)


def _mm2(base, p, root, w, b):
    return pl.pallas_call(
        _mm2_body,
        grid=(NB,),
        in_specs=[pl.BlockSpec((BN, D), lambda i: (i, 0)),
                  pl.BlockSpec((NC, BN, D), lambda i: (0, i, 0)),
                  pl.BlockSpec((D, D), lambda i: (0, 0)),
                  pl.BlockSpec((R, D, D), lambda i: (0, 0, 0)),
                  pl.BlockSpec((1, D), lambda i: (0, 0))],
        out_specs=[pl.BlockSpec((BN, D), lambda i: (i, 0)),
                   pl.BlockSpec((R, BN, D), lambda i: (0, i, 0))],
        out_shape=[jax.ShapeDtypeStruct((N, D), jnp.float32),
                   jax.ShapeDtypeStruct((R, N, D), jnp.float32)],
    )(base, p, root, w, b)


def _pool_body(base_ref, p_ref, batch_ref, linw_ref, linb_ref, out_ref,
               sums, cnts):
    i = pl.program_id(0)

    @pl.when(i == 0)
    def _():
        sums[...] = jnp.zeros((G, D), jnp.float32)
        cnts[...] = jnp.zeros((G, D), jnp.float32)

    h = jnp.maximum(base_ref[...] + p_ref[0].astype(jnp.float32)
                    + p_ref[1].astype(jnp.float32), 0.0)
    b = batch_ref[...]
    oh = (b == lax.broadcasted_iota(jnp.int32, (BN, G), 1)).astype(jnp.float32)
    sums[...] += lax.dot_general(oh, h, (((0,), (0,)), ((), ())),
                                 preferred_element_type=jnp.float32)
    cnts[...] += jnp.sum(oh, axis=0)[:, None]

    @pl.when(i == NB - 1)
    def _():
        pooled = sums[...] / jnp.maximum(cnts[...], 1.0)
        out_ref[...] = jnp.dot(pooled, linw_ref[...],
                               preferred_element_type=jnp.float32) + linb_ref[...]


def _pool(base, p, batch, linw, linb):
    return pl.pallas_call(
        _pool_body,
        grid=(NB,),
        in_specs=[pl.BlockSpec((BN, D), lambda i: (i, 0)),
                  pl.BlockSpec((NC, BN, D), lambda i: (0, i, 0)),
                  pl.BlockSpec((BN, 1), lambda i: (i, 0)),
                  pl.BlockSpec((D, D), lambda i: (0, 0)),
                  pl.BlockSpec((1, D), lambda i: (0, 0))],
        out_specs=pl.BlockSpec((G, D), lambda i: (0, 0)),
        out_shape=jax.ShapeDtypeStruct((G, D), jnp.float32),
        scratch_shapes=[pltpu.VMEM((G, D), jnp.float32),
                        pltpu.VMEM((G, D), jnp.float32)],
    )(base, p, batch, linw, linb)


def kernel(x, edge_index, edge_type, batch, W1, root1, b1, W2, root2, b2,
           linW, linb):
    src = edge_index[0].astype(jnp.int32)
    dst = edge_index[1].astype(jnp.int32)
    et = edge_type.astype(jnp.int32)
    pad = E_PAD - E
    src_p = jnp.pad(src, (0, pad))
    dst_p = jnp.pad(dst, (0, pad))
    et_p = jnp.pad(et, (0, pad))

    g_idx, w_edge = _count_weights(et_p, src_p, dst_p)
    pk2 = ((g_idx << _DBITS) | dst_p).reshape(E_PAD // CH, CH)
    wpk2 = jax.lax.bitcast_convert_type(
        w_edge.astype(jnp.bfloat16).reshape(E_PAD // 2, 2), jnp.int32
    ).reshape(E_PAD // CH, CH // 2)

    base1, y1 = _mm1(x, root1, W1, b1.reshape(1, D))
    p1 = _scatter(y1.reshape(R * N, D), pk2, wpk2)
    base2, y2 = _mm2(base1, p1, root2, W2, b2.reshape(1, D))
    p2 = _scatter(y2.reshape(R * N, D), pk2, wpk2)

    linWp = jnp.zeros((D, D), jnp.float32).at[:, :2].set(linW)
    linbp = jnp.zeros((1, D), jnp.float32).at[0, :2].set(linb)
    out = _pool(base2, p2, batch.astype(jnp.int32).reshape(N, 1),
                linWp, linbp)
    return out[:, :2]


# speedup vs baseline: 1.2216x; 1.0954x over previous
import functools

import jax
import jax.numpy as jnp
from jax import lax
from jax.experimental import pallas as pl
from jax.experimental.pallas import tpu as pltpu
from jax.experimental.pallas import tpu_sc as plsc

N = 10000
E = 320000
D = 128
R = 4
G = 8
NC = 2
NS = 16
NW = NC * NS

E_PAD = 327680
EPT = E_PAD // NW
EPC = E_PAD // NS
CH = 128
WCH = 2048
RNP = 40960
SL = RNP // NS
N_PAD = 10240
STRIPE = N_PAD // NS
BN = 400
NB = N // BN

_sc_mesh = plsc.VectorSubcoreMesh(
    core_axis_name="c", subcore_axis_name="s", num_cores=NC, num_subcores=NS)


@functools.partial(
    pl.kernel,
    out_type=[jax.ShapeDtypeStruct((E_PAD,), jnp.int32),
              jax.ShapeDtypeStruct((E_PAD,), jnp.float32)],
    mesh=_sc_mesh,
    scratch_types=[
        pltpu.VMEM_SHARED((RNP,), jnp.float32),
        pltpu.VMEM((RNP,), jnp.float32),
        pltpu.VMEM((SL,), jnp.float32),
        pltpu.VMEM((CH,), jnp.int32),
        pltpu.VMEM((CH,), jnp.int32),
        pltpu.VMEM((CH,), jnp.int32),
        pltpu.VMEM((CH,), jnp.float32),
        pltpu.VMEM((WCH,), jnp.int32),
        pltpu.VMEM((WCH,), jnp.int32),
        pltpu.VMEM((WCH,), jnp.int32),
        pltpu.VMEM((WCH,), jnp.int32),
        pltpu.VMEM((WCH,), jnp.float32),
    ],
    compiler_params=pltpu.CompilerParams(needs_layout_passes=False),
)
def _count_weights(et_hbm, src_hbm, dst_hbm, g_hbm, w_hbm,
                   c_sh, cl, zb, etb, dstb, sidxb, valb,
                   etb2, srcb2, dstb2, gb, wb):
    cid = lax.axis_index("c")
    sid = lax.axis_index("s")
    zero16 = jnp.zeros((16,), jnp.float32)
    iota16 = lax.broadcasted_iota(jnp.int32, (16,), 0)

    def _z(i, carry):
        zb[pl.ds(i * 16, 16)] = zero16
        return carry
    lax.fori_loop(0, SL // 16, _z, 0)
    pltpu.sync_copy(zb, c_sh.at[pl.ds(sid * SL, SL)])
    plsc.subcore_barrier()

    def _cchunk(j, carry):
        base = sid * EPC + j * CH
        pltpu.sync_copy(et_hbm.at[pl.ds(base, CH)], etb)
        pltpu.sync_copy(dst_hbm.at[pl.ds(base, CH)], dstb)

        def _grp(k, c2):
            o = k * 16
            s16 = etb[pl.ds(o, 16)] * N + dstb[pl.ds(o, 16)]
            v16 = jnp.where(base + o + iota16 < E, 1.0, 0.0)
            sidxb[pl.ds(o, 16)] = s16
            valb[pl.ds(o, 16)] = v16
            return c2
        lax.fori_loop(0, CH // 16, _grp, 0)
        pltpu.sync_copy(valb, c_sh.at[sidxb], add=True)
        return carry
    lax.fori_loop(0, EPC // CH, _cchunk, 0)
    plsc.subcore_barrier()
    pltpu.sync_copy(c_sh, cl)

    wid = cid * NS + sid

    def _wchunk(j, carry):
        base = wid * EPT + j * WCH
        pltpu.sync_copy(et_hbm.at[pl.ds(base, WCH)], etb2)
        pltpu.sync_copy(src_hbm.at[pl.ds(base, WCH)], srcb2)
        pltpu.sync_copy(dst_hbm.at[pl.ds(base, WCH)], dstb2)

        def _grp(k, c2):
            o = k * 16
            et16 = etb2[pl.ds(o, 16)]
            gb[pl.ds(o, 16)] = et16 * N + srcb2[pl.ds(o, 16)]
            s16 = et16 * N + dstb2[pl.ds(o, 16)]
            c16 = plsc.load_gather(cl, [s16])
            w16 = jnp.where(base + o + iota16 < E,
                            1.0 / jnp.maximum(c16, 1.0), 0.0)
            wb[pl.ds(o, 16)] = w16
            return c2
        lax.fori_loop(0, WCH // 16, _grp, 0)
        pltpu.sync_copy(gb, g_hbm.at[pl.ds(base, WCH)])
        pltpu.sync_copy(wb, w_hbm.at[pl.ds(base, WCH)])
        return carry
    lax.fori_loop(0, EPT // WCH, _wchunk, 0)


NCHT = EPT // CH
_DBITS = 14
NCH0 = 150
NCH1 = 10


@functools.partial(
    pl.kernel,
    out_type=jax.ShapeDtypeStruct((NC, N_PAD, D), jnp.float32),
    mesh=_sc_mesh,
    scratch_types=[
        pltpu.VMEM_SHARED((N_PAD, D), jnp.float32),
        pltpu.VMEM((CH,), jnp.int32),
        pltpu.VMEM((CH,), jnp.int32),
        pltpu.VMEM((CH // 2,), jnp.int32),
        pltpu.VMEM((CH // 2,), jnp.int32),
        pltpu.VMEM((CH,), jnp.int32),
        pltpu.VMEM((CH,), jnp.int32),
        pltpu.VMEM((CH,), jnp.int32),
        pltpu.VMEM((CH,), jnp.int32),
        pltpu.VMEM((CH, D), jnp.float32),
        pltpu.VMEM((CH, D), jnp.float32),
        pltpu.SemaphoreType.DMA,
        pltpu.SemaphoreType.DMA,
        pltpu.SemaphoreType.DMA,
        pltpu.SemaphoreType.DMA,
        pltpu.SemaphoreType.DMA,
        pltpu.SemaphoreType.DMA,
    ],
    compiler_params=pltpu.CompilerParams(needs_layout_passes=False),
)
def _scatter(y_hbm, pk_hbm, w_hbm, p_hbm,
             a_sh, pkr0, pkr1, wr0, wr1, gi0, gi1, di0, di1, rows0, rows1,
             sp0, sp1, sg0, sg1, ss0, ss1):
    cid = lax.axis_index("c")
    sid = lax.axis_index("s")
    zero16 = jnp.zeros((16,), jnp.float32)

    def _z(i, carry):
        rows0[i // 8, pl.ds((i % 8) * 16, 16)] = zero16
        return carry
    lax.fori_loop(0, (CH * D) // 16, _z, 0)

    def _zs(t, carry):
        pltpu.sync_copy(rows0, a_sh.at[pl.ds(sid * STRIPE + t * CH, CH)])
        return carry
    lax.fori_loop(0, STRIPE // CH, _zs, 0)
    plsc.subcore_barrier()

    ncht = jnp.where(cid == 0, NCH0, NCH1)
    cb = jnp.where(cid == 0, sid * NCH0, NS * NCH0 + sid * NCH1)

    def _unpack(pkr_s, gi_s, di_s):
        for k in range(CH // 16):
            o = k * 16
            p16 = pkr_s[pl.ds(o, 16)]
            gi_s[pl.ds(o, 16)] = lax.shift_right_logical(p16, _DBITS)
            di_s[pl.ds(o, 16)] = lax.bitwise_and(p16, (1 << _DBITS) - 1)

    bufs = ((rows0, pkr0, wr0, gi0, di0, sp0, sg0, ss0),
            (rows1, pkr1, wr1, gi1, di1, sp1, sg1, ss1))

    pltpu.sync_copy(pk_hbm.at[cb], pkr0)
    pltpu.sync_copy(w_hbm.at[cb], wr0)
    _unpack(pkr0, gi0, di0)
    pltpu.async_copy(y_hbm.at[gi0], rows0, sg0)
    pltpu.async_copy(pk_hbm.at[cb + 1], pkr1, sp1)
    pltpu.async_copy(w_hbm.at[cb + 1], wr1, sp1)

    def _pair(jj, carry):
        for b in range(2):
            j = jj * 2 + b
            rows_b, pkr_b, wr_b, gi_b, di_b, sp_b, sg_b, ss_b = bufs[b]
            rows_n, pkr_n, wr_n, gi_n, di_n, sp_n, sg_n, ss_n = bufs[1 - b]

            @pl.when(j + 1 < ncht)
            def _():
                @pl.when(j >= 1)
                def _():
                    pltpu.make_async_copy(rows_n, a_sh.at[di_n], ss_n).wait()
                pltpu.make_async_copy(pk_hbm.at[cb], pkr_n, sp_n).wait()
                pltpu.make_async_copy(w_hbm.at[cb], wr_n, sp_n).wait()
                _unpack(pkr_n, gi_n, di_n)
                pltpu.async_copy(y_hbm.at[gi_n], rows_n, sg_n)

            pltpu.make_async_copy(y_hbm.at[gi_b], rows_b, sg_b).wait()

            def _scale(k, c2):
                wp16 = wr_b[pl.ds(k * 16, 16)]
                we = plsc.bitcast(lax.shift_left(wp16, 16), jnp.float32)
                wo = plsc.bitcast(lax.bitwise_and(wp16, jnp.int32(-65536)),
                                  jnp.float32)
                for l in range(16):
                    e0 = k * 32 + 2 * l
                    wl0 = jnp.full((16,), we[l], jnp.float32)
                    wl1 = jnp.full((16,), wo[l], jnp.float32)
                    for c in range(D // 16):
                        rows_b[e0, pl.ds(c * 16, 16)] = (
                            rows_b[e0, pl.ds(c * 16, 16)] * wl0)
                        rows_b[e0 + 1, pl.ds(c * 16, 16)] = (
                            rows_b[e0 + 1, pl.ds(c * 16, 16)] * wl1)
                return c2
            lax.fori_loop(0, CH // 32, _scale, 0)
            pltpu.async_copy(rows_b, a_sh.at[di_b], ss_b, add=True)

            @pl.when(j + 2 < ncht)
            def _():
                pltpu.async_copy(pk_hbm.at[cb + j + 2], pkr_b, sp_b)
                pltpu.async_copy(w_hbm.at[cb + j + 2], wr_b, sp_b)
        return carry
    lax.fori_loop(0, lax.div(ncht, 2), _pair, 0)
    pltpu.make_async_copy(rows0, a_sh.at[di0], ss0).wait()
    pltpu.make_async_copy(rows1, a_sh.at[di1], ss1).wait()
    plsc.subcore_barrier()

    def _out(t, carry):
        off = sid * STRIPE + t * CH
        pltpu.sync_copy(a_sh.at[pl.ds(off, CH)], p_hbm.at[cid, pl.ds(off, CH)])
        return carry
    lax.fori_loop(0, STRIPE // CH, _out, 0)


def _mm1_body(x_ref, root_ref, w_ref, b_ref, base_ref, y_ref):
    xb = x_ref[...]
    base_ref[...] = jnp.dot(xb, root_ref[...],
                            preferred_element_type=jnp.float32) + b_ref[...]
    for r in range(R):
        y_ref[r] = jnp.dot(xb, w_ref[r], preferred_element_type=jnp.float32)


def _mm1(x, root, w, b):
    return pl.pallas_call(
        _mm1_body,
        grid=(NB,),
        in_specs=[pl.BlockSpec((BN, D), lambda i: (i, 0)),
                  pl.BlockSpec((D, D), lambda i: (0, 0)),
                  pl.BlockSpec((R, D, D), lambda i: (0, 0, 0)),
                  pl.BlockSpec((1, D), lambda i: (0, 0))],
        out_specs=[pl.BlockSpec((BN, D), lambda i: (i, 0)),
                   pl.BlockSpec((R, BN, D), lambda i: (0, i, 0))],
        out_shape=[jax.ShapeDtypeStruct((N, D), jnp.float32),
                   jax.ShapeDtypeStruct((R, N, D), jnp.float32)],
    )(x, root, w, b)


def _mm2_body(base_ref, p_ref, root_ref, w_ref, b_ref, base2_ref, y_ref):
    h = jnp.maximum(base_ref[...] + p_ref[0].astype(jnp.float32)
                    + p_ref[1].astype(jnp.float32), 0.0)
    base2_ref[...] = jnp.dot(h, root_ref[...],
                             preferred_element_type=jnp.float32) + b_ref[...]
    for r in range(R):
        y_ref[r] = jnp.dot(h, w_ref[r], preferred_element_type=jnp.float32)


def _mm2(base, p, root, w, b):
    return pl.pallas_call(
        _mm2_body,
        grid=(NB,),
        in_specs=[pl.BlockSpec((BN, D), lambda i: (i, 0)),
                  pl.BlockSpec((NC, BN, D), lambda i: (0, i, 0)),
                  pl.BlockSpec((D, D), lambda i: (0, 0)),
                  pl.BlockSpec((R, D, D), lambda i: (0, 0, 0)),
                  pl.BlockSpec((1, D), lambda i: (0, 0))],
        out_specs=[pl.BlockSpec((BN, D), lambda i: (i, 0)),
                   pl.BlockSpec((R, BN, D), lambda i: (0, i, 0))],
        out_shape=[jax.ShapeDtypeStruct((N, D), jnp.float32),
                   jax.ShapeDtypeStruct((R, N, D), jnp.float32)],
    )(base, p, root, w, b)


def _pool_body(base_ref, p_ref, batch_ref, linw_ref, linb_ref, out_ref,
               sums, cnts):
    i = pl.program_id(0)

    @pl.when(i == 0)
    def _():
        sums[...] = jnp.zeros((G, D), jnp.float32)
        cnts[...] = jnp.zeros((G, D), jnp.float32)

    h = jnp.maximum(base_ref[...] + p_ref[0].astype(jnp.float32)
                    + p_ref[1].astype(jnp.float32), 0.0)
    b = batch_ref[...]
    oh = (b == lax.broadcasted_iota(jnp.int32, (BN, G), 1)).astype(jnp.float32)
    sums[...] += lax.dot_general(oh, h, (((0,), (0,)), ((), ())),
                                 preferred_element_type=jnp.float32)
    cnts[...] += jnp.sum(oh, axis=0)[:, None]

    @pl.when(i == NB - 1)
    def _():
        pooled = sums[...] / jnp.maximum(cnts[...], 1.0)
        out_ref[...] = jnp.dot(pooled, linw_ref[...],
                               preferred_element_type=jnp.float32) + linb_ref[...]


def _pool(base, p, batch, linw, linb):
    return pl.pallas_call(
        _pool_body,
        grid=(NB,),
        in_specs=[pl.BlockSpec((BN, D), lambda i: (i, 0)),
                  pl.BlockSpec((NC, BN, D), lambda i: (0, i, 0)),
                  pl.BlockSpec((BN, 1), lambda i: (i, 0)),
                  pl.BlockSpec((D, D), lambda i: (0, 0)),
                  pl.BlockSpec((1, D), lambda i: (0, 0))],
        out_specs=pl.BlockSpec((G, D), lambda i: (0, 0)),
        out_shape=jax.ShapeDtypeStruct((G, D), jnp.float32),
        scratch_shapes=[pltpu.VMEM((G, D), jnp.float32),
                        pltpu.VMEM((G, D), jnp.float32)],
    )(base, p, batch, linw, linb)


def kernel(x, edge_index, edge_type, batch, W1, root1, b1, W2, root2, b2,
           linW, linb):
    src = edge_index[0].astype(jnp.int32)
    dst = edge_index[1].astype(jnp.int32)
    et = edge_type.astype(jnp.int32)
    pad = E_PAD - E
    src_p = jnp.pad(src, (0, pad))
    dst_p = jnp.pad(dst, (0, pad))
    et_p = jnp.pad(et, (0, pad))

    g_idx, w_edge = _count_weights(et_p, src_p, dst_p)
    pk2 = ((g_idx << _DBITS) | dst_p).reshape(E_PAD // CH, CH)
    wpk2 = jax.lax.bitcast_convert_type(
        w_edge.astype(jnp.bfloat16).reshape(E_PAD // 2, 2), jnp.int32
    ).reshape(E_PAD // CH, CH // 2)

    base1, y1 = _mm1(x, root1, W1, b1.reshape(1, D))
    p1 = _scatter(y1.reshape(R * N, D), pk2, wpk2)
    base2, y2 = _mm2(base1, p1, root2, W2, b2.reshape(1, D))
    p2 = _scatter(y2.reshape(R * N, D), pk2, wpk2)

    linWp = jnp.zeros((D, D), jnp.float32).at[:, :2].set(linW)
    linbp = jnp.zeros((1, D), jnp.float32).at[0, :2].set(linb)
    out = _pool(base2, p2, batch.astype(jnp.int32).reshape(N, 1),
                linWp, linbp)
    return out[:, :2]


# speedup vs baseline: 1.3369x; 1.0943x over previous
import functools

import jax
import jax.numpy as jnp
from jax import lax
from jax.experimental import pallas as pl
from jax.experimental.pallas import tpu as pltpu
from jax.experimental.pallas import tpu_sc as plsc

N = 10000
E = 320000
D = 128
DW = D // 2
R = 4
G = 8
NC = 2
NS = 16
NW = NC * NS

E_PAD = 327680
NCHG = E_PAD // 128
EPT = E_PAD // NW
EPC = E_PAD // NS
CH = 128
WCH = 2048
RNP = 40960
SL = RNP // NS
N_PAD = 10112
ZCH = 128
BN = 400
NB = N // BN
_DBITS = 14
NCH0 = 96
NCH1 = 64

LO_SEL = [32 * (c // 16) + (c % 16) for c in range(DW)]
HI_SEL = [32 * (c // 16) + 16 + (c % 16) for c in range(DW)]

_sc_mesh = plsc.VectorSubcoreMesh(
    core_axis_name="c", subcore_axis_name="s", num_cores=NC, num_subcores=NS)


@functools.partial(
    pl.kernel,
    out_type=[jax.ShapeDtypeStruct((NCHG, CH), jnp.int32),
              jax.ShapeDtypeStruct((NCHG, CH // 2), jnp.int32)],
    mesh=_sc_mesh,
    scratch_types=[
        pltpu.VMEM_SHARED((RNP,), jnp.float32),
        pltpu.VMEM((RNP,), jnp.float32),
        pltpu.VMEM((SL,), jnp.float32),
        pltpu.VMEM((CH,), jnp.int32),
        pltpu.VMEM((CH,), jnp.int32),
        pltpu.VMEM((CH,), jnp.int32),
        pltpu.VMEM((CH,), jnp.float32),
        pltpu.VMEM((WCH,), jnp.int32),
        pltpu.VMEM((WCH,), jnp.int32),
        pltpu.VMEM((WCH,), jnp.int32),
        pltpu.VMEM((WCH // CH, CH), jnp.int32),
        pltpu.VMEM((WCH // CH, CH // 2), jnp.int32),
    ],
    compiler_params=pltpu.CompilerParams(needs_layout_passes=False),
)
def _count_weights(et_hbm, src_hbm, dst_hbm, pk_hbm, wpk_hbm,
                   c_sh, cl, zb, etb, dstb, sidxb, valb,
                   etb2, srcb2, dstb2, gb2, wb2):
    cid = lax.axis_index("c")
    sid = lax.axis_index("s")
    zero16 = jnp.zeros((16,), jnp.float32)
    iota16 = lax.broadcasted_iota(jnp.int32, (16,), 0)

    def _z(i, carry):
        zb[pl.ds(i * 16, 16)] = zero16
        return carry
    lax.fori_loop(0, SL // 16, _z, 0)
    pltpu.sync_copy(zb, c_sh.at[pl.ds(sid * SL, SL)])
    plsc.subcore_barrier()

    def _cchunk(j, carry):
        base = sid * EPC + j * CH
        pltpu.sync_copy(et_hbm.at[pl.ds(base, CH)], etb)
        pltpu.sync_copy(dst_hbm.at[pl.ds(base, CH)], dstb)

        def _grp(k, c2):
            o = k * 16
            s16 = etb[pl.ds(o, 16)] * N + dstb[pl.ds(o, 16)]
            v16 = jnp.where(base + o + iota16 < E, 1.0, 0.0)
            sidxb[pl.ds(o, 16)] = s16
            valb[pl.ds(o, 16)] = v16
            return c2
        lax.fori_loop(0, CH // 16, _grp, 0)
        pltpu.sync_copy(valb, c_sh.at[sidxb], add=True)
        return carry
    lax.fori_loop(0, EPC // CH, _cchunk, 0)
    plsc.subcore_barrier()
    pltpu.sync_copy(c_sh, cl)

    wid = cid * NS + sid

    def _w16(base, o):
        et16 = etb2[pl.ds(o, 16)]
        d16 = dstb2[pl.ds(o, 16)]
        g16 = et16 * N + srcb2[pl.ds(o, 16)]
        c16 = plsc.load_gather(cl, [et16 * N + d16])
        w16 = jnp.where(base + o + iota16 < E,
                        1.0 / jnp.maximum(c16, 1.0), 0.0)
        return lax.shift_left(g16, _DBITS) | d16, w16

    def _wchunk(j, carry):
        base = wid * EPT + j * WCH
        pltpu.sync_copy(et_hbm.at[pl.ds(base, WCH)], etb2)
        pltpu.sync_copy(src_hbm.at[pl.ds(base, WCH)], srcb2)
        pltpu.sync_copy(dst_hbm.at[pl.ds(base, WCH)], dstb2)

        def _grp(k, c2):
            pk16a, wa = _w16(base, k * 32)
            pk16b, wb = _w16(base, k * 32 + 16)
            row = k // 4
            col = (k % 4) * 32
            gb2[row, pl.ds(col, 16)] = pk16a
            gb2[row, pl.ds(col + 16, 16)] = pk16b
            wword = lax.bitwise_or(
                lax.shift_right_logical(plsc.bitcast(wa, jnp.int32), 16),
                lax.bitwise_and(plsc.bitcast(wb, jnp.int32),
                                jnp.int32(-65536)))
            wb2[row, pl.ds((k % 4) * 16, 16)] = wword
            return c2
        lax.fori_loop(0, WCH // 32, _grp, 0)
        rows0 = wid * (EPT // CH) + j * (WCH // CH)
        pltpu.sync_copy(gb2, pk_hbm.at[pl.ds(rows0, WCH // CH)])
        pltpu.sync_copy(wb2, wpk_hbm.at[pl.ds(rows0, WCH // CH)])
        return carry
    lax.fori_loop(0, EPT // WCH, _wchunk, 0)


@functools.partial(
    pl.kernel,
    out_type=jax.ShapeDtypeStruct((NC, N_PAD, D), jnp.float32),
    mesh=_sc_mesh,
    scratch_types=[
        pltpu.VMEM_SHARED((N_PAD, D), jnp.float32),
        pltpu.VMEM((CH // 2,), jnp.int32),
        pltpu.VMEM((CH // 2,), jnp.int32),
        pltpu.VMEM((CH,), jnp.int32),
        pltpu.VMEM((CH,), jnp.int32),
        pltpu.VMEM((CH,), jnp.int32),
        pltpu.VMEM((CH,), jnp.int32),
        pltpu.VMEM((CH, DW), jnp.int32),
        pltpu.VMEM((CH, DW), jnp.int32),
        pltpu.VMEM((CH, D), jnp.float32),
        pltpu.VMEM((CH, D), jnp.float32),
        pltpu.SemaphoreType.DMA,
        pltpu.SemaphoreType.DMA,
        pltpu.SemaphoreType.DMA,
        pltpu.SemaphoreType.DMA,
        pltpu.SemaphoreType.DMA,
        pltpu.SemaphoreType.DMA,
    ],
    compiler_params=pltpu.CompilerParams(needs_layout_passes=False,
                                          use_tc_tiling_on_sc=False),
)
def _scatter(y_hbm, pk_hbm, w_hbm, p_hbm,
             a_sh, wr0, wr1, gi0, gi1, di0, di1,
             rows0, rows1, frows0, frows1, sp0, sp1, sg0, sg1, ss0, ss1):
    cid = lax.axis_index("c")
    sid = lax.axis_index("s")
    zero16 = jnp.zeros((16,), jnp.float32)
    m16 = jnp.int32(-65536)

    def _z(i, carry):
        frows0[i // 8, pl.ds((i % 8) * 16, 16)] = zero16
        return carry
    lax.fori_loop(0, (CH * D) // 16, _z, 0)

    nstr = jnp.where(sid == NS - 1, 4, 5)

    def _zs(t, carry):
        pltpu.sync_copy(frows0, a_sh.at[pl.ds(sid * 640 + t * ZCH, ZCH)])
        return carry
    lax.fori_loop(0, nstr, _zs, 0)
    plsc.subcore_barrier()

    ncht = jnp.where(cid == 0, NCH0, NCH1)
    cb = jnp.where(cid == 0, sid * NCH0, NS * NCH0 + sid * NCH1)

    def _unpack(gi_s, di_s):
        for k in range(CH // 16):
            o = k * 16
            p16 = gi_s[pl.ds(o, 16)]
            di_s[pl.ds(o, 16)] = lax.bitwise_and(p16, (1 << _DBITS) - 1)
            gi_s[pl.ds(o, 16)] = lax.shift_right_logical(p16, _DBITS)

    bufs = ((rows0, frows0, wr0, gi0, di0, sp0, sg0, ss0),
            (rows1, frows1, wr1, gi1, di1, sp1, sg1, ss1))

    pltpu.sync_copy(pk_hbm.at[cb], gi0)
    pltpu.sync_copy(w_hbm.at[cb], wr0)
    _unpack(gi0, di0)
    pltpu.async_copy(y_hbm.at[gi0], rows0, sg0)
    pltpu.async_copy(pk_hbm.at[cb + 1], gi1, sp1)
    pltpu.async_copy(w_hbm.at[cb + 1], wr1, sp1)

    def _pair(jj, carry):
        for b in range(2):
            j = jj * 2 + b
            rows_b, frows_b, wr_b, gi_b, di_b, sp_b, sg_b, ss_b = bufs[b]
            rows_n, frows_n, wr_n, gi_n, di_n, sp_n, sg_n, ss_n = bufs[1 - b]

            @pl.when(j + 1 < ncht)
            def _():
                @pl.when(j >= 1)
                def _():
                    pltpu.make_async_copy(frows_n, a_sh.at[di_n],
                                          ss_n).wait()
                pltpu.make_async_copy(pk_hbm.at[cb], gi_n, sp_n).wait()
                pltpu.make_async_copy(w_hbm.at[cb], wr_n, sp_n).wait()
                _unpack(gi_n, di_n)
                pltpu.async_copy(y_hbm.at[gi_n], rows_n, sg_n)

            pltpu.make_async_copy(y_hbm.at[gi_b], rows_b, sg_b).wait()

            def _scale(k, c2):
                wp16 = wr_b[pl.ds(k * 16, 16)]
                we = plsc.bitcast(lax.shift_left(wp16, 16), jnp.float32)
                wo = plsc.bitcast(lax.bitwise_and(wp16, m16), jnp.float32)
                for l in range(16):
                    for (e, wl16) in ((k * 32 + l, we), (k * 32 + 16 + l, wo)):
                        wl = jnp.full((16,), wl16[l], jnp.float32)
                        for c in range(DW // 16):
                            wrd = rows_b[e, pl.ds(c * 16, 16)]
                            lo = plsc.bitcast(lax.shift_left(wrd, 16),
                                              jnp.float32)
                            hi = plsc.bitcast(lax.bitwise_and(wrd, m16),
                                              jnp.float32)
                            frows_b[e, pl.ds(c * 32, 16)] = lo * wl
                            frows_b[e, pl.ds(c * 32 + 16, 16)] = hi * wl
                return c2
            lax.fori_loop(0, CH // 32, _scale, 0)
            pltpu.async_copy(frows_b, a_sh.at[di_b], ss_b, add=True)

            @pl.when(j + 2 < ncht)
            def _():
                pltpu.async_copy(pk_hbm.at[cb + j + 2], gi_b, sp_b)
                pltpu.async_copy(w_hbm.at[cb + j + 2], wr_b, sp_b)
        return carry
    lax.fori_loop(0, lax.div(ncht, 2), _pair, 0)
    pltpu.make_async_copy(frows0, a_sh.at[di0], ss0).wait()
    pltpu.make_async_copy(frows1, a_sh.at[di1], ss1).wait()
    plsc.subcore_barrier()

    def _out(t, carry):
        off = sid * 640 + t * ZCH
        pltpu.sync_copy(a_sh.at[pl.ds(off, ZCH)],
                        p_hbm.at[cid, pl.ds(off, ZCH)])
        return carry
    lax.fori_loop(0, nstr, _out, 0)


def _pack_y(h, wlo_r, whi_r):
    ylo = jnp.dot(h, wlo_r, preferred_element_type=jnp.float32
                  ).astype(jnp.bfloat16)
    yhi = jnp.dot(h, whi_r, preferred_element_type=jnp.float32
                  ).astype(jnp.bfloat16)
    lo_i = lax.bitcast_convert_type(ylo, jnp.uint16).astype(jnp.int32)
    hi_i = lax.bitcast_convert_type(yhi, jnp.uint16).astype(jnp.int32)
    return lo_i | lax.shift_left(hi_i, 16)


def _mm1_body(x_ref, root_ref, wlo_ref, whi_ref, b_ref, base_ref, y_ref):
    xb = x_ref[...]
    base_ref[...] = jnp.dot(xb, root_ref[...],
                            preferred_element_type=jnp.float32) + b_ref[...]
    for r in range(R):
        y_ref[r] = _pack_y(xb, wlo_ref[r], whi_ref[r])


def _mm1(x, root, wlo, whi, b):
    return pl.pallas_call(
        _mm1_body,
        grid=(NB,),
        in_specs=[pl.BlockSpec((BN, D), lambda i: (i, 0)),
                  pl.BlockSpec((D, D), lambda i: (0, 0)),
                  pl.BlockSpec((R, D, DW), lambda i: (0, 0, 0)),
                  pl.BlockSpec((R, D, DW), lambda i: (0, 0, 0)),
                  pl.BlockSpec((1, D), lambda i: (0, 0))],
        out_specs=[pl.BlockSpec((BN, D), lambda i: (i, 0)),
                   pl.BlockSpec((R, BN, DW), lambda i: (0, i, 0))],
        out_shape=[jax.ShapeDtypeStruct((N, D), jnp.float32),
                   jax.ShapeDtypeStruct((R, N, DW), jnp.int32)],
    )(x, root, wlo, whi, b)


def _mm2_body(base_ref, p_ref, root_ref, wlo_ref, whi_ref, b_ref,
              base2_ref, y_ref):
    h = jnp.maximum(base_ref[...] + p_ref[0] + p_ref[1], 0.0)
    base2_ref[...] = jnp.dot(h, root_ref[...],
                             preferred_element_type=jnp.float32) + b_ref[...]
    for r in range(R):
        y_ref[r] = _pack_y(h, wlo_ref[r], whi_ref[r])


def _mm2(base, p, root, wlo, whi, b):
    return pl.pallas_call(
        _mm2_body,
        grid=(NB,),
        in_specs=[pl.BlockSpec((BN, D), lambda i: (i, 0)),
                  pl.BlockSpec((NC, BN, D), lambda i: (0, i, 0)),
                  pl.BlockSpec((D, D), lambda i: (0, 0)),
                  pl.BlockSpec((R, D, DW), lambda i: (0, 0, 0)),
                  pl.BlockSpec((R, D, DW), lambda i: (0, 0, 0)),
                  pl.BlockSpec((1, D), lambda i: (0, 0))],
        out_specs=[pl.BlockSpec((BN, D), lambda i: (i, 0)),
                   pl.BlockSpec((R, BN, DW), lambda i: (0, i, 0))],
        out_shape=[jax.ShapeDtypeStruct((N, D), jnp.float32),
                   jax.ShapeDtypeStruct((R, N, DW), jnp.int32)],
    )(base, p, root, wlo, whi, b)


def _pool_body(base_ref, p_ref, batch_ref, linw_ref, linb_ref, out_ref,
               sums, cnts):
    i = pl.program_id(0)

    @pl.when(i == 0)
    def _():
        sums[...] = jnp.zeros((G, D), jnp.float32)
        cnts[...] = jnp.zeros((G, D), jnp.float32)

    h = jnp.maximum(base_ref[...] + p_ref[0] + p_ref[1], 0.0)
    b = batch_ref[...]
    oh = (b == lax.broadcasted_iota(jnp.int32, (BN, G), 1)).astype(jnp.float32)
    sums[...] += lax.dot_general(oh, h, (((0,), (0,)), ((), ())),
                                 preferred_element_type=jnp.float32)
    cnts[...] += jnp.sum(oh, axis=0)[:, None]

    @pl.when(i == NB - 1)
    def _():
        pooled = sums[...] / jnp.maximum(cnts[...], 1.0)
        out_ref[...] = jnp.dot(pooled, linw_ref[...],
                               preferred_element_type=jnp.float32) + linb_ref[...]


def _pool(base, p, batch, linw, linb):
    return pl.pallas_call(
        _pool_body,
        grid=(NB,),
        in_specs=[pl.BlockSpec((BN, D), lambda i: (i, 0)),
                  pl.BlockSpec((NC, BN, D), lambda i: (0, i, 0)),
                  pl.BlockSpec((BN, 1), lambda i: (i, 0)),
                  pl.BlockSpec((D, D), lambda i: (0, 0)),
                  pl.BlockSpec((1, D), lambda i: (0, 0))],
        out_specs=pl.BlockSpec((G, D), lambda i: (0, 0)),
        out_shape=jax.ShapeDtypeStruct((G, D), jnp.float32),
        scratch_shapes=[pltpu.VMEM((G, D), jnp.float32),
                        pltpu.VMEM((G, D), jnp.float32)],
    )(base, p, batch, linw, linb)


def kernel(x, edge_index, edge_type, batch, W1, root1, b1, W2, root2, b2,
           linW, linb):
    src = edge_index[0].astype(jnp.int32)
    dst = edge_index[1].astype(jnp.int32)
    et = edge_type.astype(jnp.int32)
    pad = E_PAD - E
    src_p = jnp.pad(src, (0, pad))
    dst_p = jnp.pad(dst, (0, pad))
    et_p = jnp.pad(et, (0, pad))

    pk2, wpk2 = _count_weights(et_p, src_p, dst_p)

    lo_sel = jnp.array(LO_SEL, dtype=jnp.int32)
    hi_sel = jnp.array(HI_SEL, dtype=jnp.int32)
    w1lo = W1[:, :, lo_sel]
    w1hi = W1[:, :, hi_sel]
    w2lo = W2[:, :, lo_sel]
    w2hi = W2[:, :, hi_sel]

    base1, y1 = _mm1(x, root1, w1lo, w1hi, b1.reshape(1, D))
    p1 = _scatter(y1.reshape(R * N, DW), pk2, wpk2)
    base2, y2 = _mm2(base1, p1, root2, w2lo, w2hi, b2.reshape(1, D))
    p2 = _scatter(y2.reshape(R * N, DW), pk2, wpk2)

    linWp = jnp.zeros((D, D), jnp.float32).at[:, :2].set(linW)
    linbp = jnp.zeros((1, D), jnp.float32).at[0, :2].set(linb)
    out = _pool(base2, p2, batch.astype(jnp.int32).reshape(N, 1),
                linWp, linbp)
    return out[:, :2]


# speedup vs baseline: 1.4880x; 1.1131x over previous
import functools

import jax
import jax.numpy as jnp
from jax import lax
from jax.experimental import pallas as pl
from jax.experimental.pallas import tpu as pltpu
from jax.experimental.pallas import tpu_sc as plsc

N = 10000
E = 320000
D = 128
DW = D // 2
R = 4
G = 8
NC = 2
NS = 16
NW = NC * NS

E_PAD = 327680
NCHG = E_PAD // 128
EPT = E_PAD // NW
EPC = E_PAD // NS
CH = 128
WCH = 2048
RNP = 40960
SL = RNP // NS
N_PAD = 10112
ZCH = 128
BN = 400
NB = N // BN
_DBITS = 14
NCH0 = 80
NCH1 = 80

LO_SEL = [32 * (c // 16) + (c % 16) for c in range(DW)]
HI_SEL = [32 * (c // 16) + 16 + (c % 16) for c in range(DW)]

_sc_mesh = plsc.VectorSubcoreMesh(
    core_axis_name="c", subcore_axis_name="s", num_cores=NC, num_subcores=NS)


@functools.partial(
    pl.kernel,
    out_type=[jax.ShapeDtypeStruct((NCHG, CH), jnp.int32),
              jax.ShapeDtypeStruct((NCHG, CH // 2), jnp.int32)],
    mesh=_sc_mesh,
    scratch_types=[
        pltpu.VMEM_SHARED((RNP,), jnp.float32),
        pltpu.VMEM((RNP,), jnp.float32),
        pltpu.VMEM((SL,), jnp.float32),
        pltpu.VMEM((CH,), jnp.int32),
        pltpu.VMEM((CH,), jnp.int32),
        pltpu.VMEM((CH,), jnp.int32),
        pltpu.VMEM((CH,), jnp.float32),
        pltpu.VMEM((WCH,), jnp.int32),
        pltpu.VMEM((WCH,), jnp.int32),
        pltpu.VMEM((WCH,), jnp.int32),
        pltpu.VMEM((WCH // CH, CH), jnp.int32),
        pltpu.VMEM((WCH // CH, CH // 2), jnp.int32),
    ],
    compiler_params=pltpu.CompilerParams(needs_layout_passes=False),
)
def _count_weights(et_hbm, src_hbm, dst_hbm, pk_hbm, wpk_hbm,
                   c_sh, cl, zb, etb, dstb, sidxb, valb,
                   etb2, srcb2, dstb2, gb2, wb2):
    cid = lax.axis_index("c")
    sid = lax.axis_index("s")
    zero16 = jnp.zeros((16,), jnp.float32)
    iota16 = lax.broadcasted_iota(jnp.int32, (16,), 0)

    def _z(i, carry):
        zb[pl.ds(i * 16, 16)] = zero16
        return carry
    lax.fori_loop(0, SL // 16, _z, 0)
    pltpu.sync_copy(zb, c_sh.at[pl.ds(sid * SL, SL)])
    plsc.subcore_barrier()

    def _cchunk(j, carry):
        base = sid * EPC + j * CH
        pltpu.sync_copy(et_hbm.at[pl.ds(base, CH)], etb)
        pltpu.sync_copy(dst_hbm.at[pl.ds(base, CH)], dstb)

        def _grp(k, c2):
            o = k * 16
            s16 = etb[pl.ds(o, 16)] * N + dstb[pl.ds(o, 16)]
            v16 = jnp.where(base + o + iota16 < E, 1.0, 0.0)
            sidxb[pl.ds(o, 16)] = s16
            valb[pl.ds(o, 16)] = v16
            return c2
        lax.fori_loop(0, CH // 16, _grp, 0)
        pltpu.sync_copy(valb, c_sh.at[sidxb], add=True)
        return carry
    lax.fori_loop(0, EPC // CH, _cchunk, 0)
    plsc.subcore_barrier()
    pltpu.sync_copy(c_sh, cl)

    wid = cid * NS + sid

    def _w16(base, o):
        et16 = etb2[pl.ds(o, 16)]
        d16 = dstb2[pl.ds(o, 16)]
        g16 = et16 * N + srcb2[pl.ds(o, 16)]
        c16 = plsc.load_gather(cl, [et16 * N + d16])
        w16 = jnp.where(base + o + iota16 < E,
                        1.0 / jnp.maximum(c16, 1.0), 0.0)
        return lax.shift_left(g16, _DBITS) | d16, w16

    def _wchunk(j, carry):
        base = wid * EPT + j * WCH
        pltpu.sync_copy(et_hbm.at[pl.ds(base, WCH)], etb2)
        pltpu.sync_copy(src_hbm.at[pl.ds(base, WCH)], srcb2)
        pltpu.sync_copy(dst_hbm.at[pl.ds(base, WCH)], dstb2)

        def _grp(k, c2):
            pk16a, wa = _w16(base, k * 32)
            pk16b, wb = _w16(base, k * 32 + 16)
            row = k // 4
            col = (k % 4) * 32
            gb2[row, pl.ds(col, 16)] = pk16a
            gb2[row, pl.ds(col + 16, 16)] = pk16b
            wword = lax.bitwise_or(
                lax.shift_right_logical(plsc.bitcast(wa, jnp.int32), 16),
                lax.bitwise_and(plsc.bitcast(wb, jnp.int32),
                                jnp.int32(-65536)))
            wb2[row, pl.ds((k % 4) * 16, 16)] = wword
            return c2
        lax.fori_loop(0, WCH // 32, _grp, 0)
        rows0 = wid * (EPT // CH) + j * (WCH // CH)
        pltpu.sync_copy(gb2, pk_hbm.at[pl.ds(rows0, WCH // CH)])
        pltpu.sync_copy(wb2, wpk_hbm.at[pl.ds(rows0, WCH // CH)])
        return carry
    lax.fori_loop(0, EPT // WCH, _wchunk, 0)


@functools.partial(
    pl.kernel,
    out_type=jax.ShapeDtypeStruct((NC, N_PAD, D), jnp.float32),
    mesh=_sc_mesh,
    scratch_types=[
        pltpu.VMEM_SHARED((N_PAD, D), jnp.float32),
        pltpu.VMEM((CH // 2,), jnp.int32),
        pltpu.VMEM((CH // 2,), jnp.int32),
        pltpu.VMEM((CH,), jnp.int32),
        pltpu.VMEM((CH,), jnp.int32),
        pltpu.VMEM((CH,), jnp.int32),
        pltpu.VMEM((CH,), jnp.int32),
        pltpu.VMEM((CH, DW), jnp.int32),
        pltpu.VMEM((CH, DW), jnp.int32),
        pltpu.VMEM((CH, D), jnp.float32),
        pltpu.VMEM((CH, D), jnp.float32),
        pltpu.SemaphoreType.DMA,
        pltpu.SemaphoreType.DMA,
        pltpu.SemaphoreType.DMA,
        pltpu.SemaphoreType.DMA,
        pltpu.SemaphoreType.DMA,
        pltpu.SemaphoreType.DMA,
    ],
    compiler_params=pltpu.CompilerParams(needs_layout_passes=False,
                                          use_tc_tiling_on_sc=False),
)
def _scatter(y_hbm, pk_hbm, w_hbm, p_hbm,
             a_sh, wr0, wr1, gi0, gi1, di0, di1,
             rows0, rows1, frows0, frows1, sp0, sp1, sg0, sg1, ss0, ss1):
    cid = lax.axis_index("c")
    sid = lax.axis_index("s")
    zero16 = jnp.zeros((16,), jnp.float32)
    m16 = jnp.int32(-65536)

    def _z(i, carry):
        frows0[i // 8, pl.ds((i % 8) * 16, 16)] = zero16
        return carry
    lax.fori_loop(0, (CH * D) // 16, _z, 0)

    nstr = jnp.where(sid == NS - 1, 4, 5)

    def _zs(t, carry):
        pltpu.sync_copy(frows0, a_sh.at[pl.ds(sid * 640 + t * ZCH, ZCH)])
        return carry
    lax.fori_loop(0, nstr, _zs, 0)
    plsc.subcore_barrier()

    ncht = jnp.where(cid == 0, NCH0, NCH1)
    cb = jnp.where(cid == 0, sid * NCH0, NS * NCH0 + sid * NCH1)

    def _unpack(gi_s, di_s):
        for k in range(CH // 16):
            o = k * 16
            p16 = gi_s[pl.ds(o, 16)]
            di_s[pl.ds(o, 16)] = lax.bitwise_and(p16, (1 << _DBITS) - 1)
            gi_s[pl.ds(o, 16)] = lax.shift_right_logical(p16, _DBITS)

    bufs = ((rows0, frows0, wr0, gi0, di0, sp0, sg0, ss0),
            (rows1, frows1, wr1, gi1, di1, sp1, sg1, ss1))

    pltpu.sync_copy(pk_hbm.at[cb], gi0)
    pltpu.sync_copy(w_hbm.at[cb], wr0)
    _unpack(gi0, di0)
    pltpu.async_copy(y_hbm.at[gi0], rows0, sg0)
    pltpu.async_copy(pk_hbm.at[cb + 1], gi1, sp1)
    pltpu.async_copy(w_hbm.at[cb + 1], wr1, sp1)

    def _pair(jj, carry):
        for b in range(2):
            j = jj * 2 + b
            rows_b, frows_b, wr_b, gi_b, di_b, sp_b, sg_b, ss_b = bufs[b]
            rows_n, frows_n, wr_n, gi_n, di_n, sp_n, sg_n, ss_n = bufs[1 - b]

            @pl.when(j + 1 < ncht)
            def _():
                @pl.when(j >= 1)
                def _():
                    pltpu.make_async_copy(frows_n, a_sh.at[di_n],
                                          ss_n).wait()
                pltpu.make_async_copy(pk_hbm.at[cb], gi_n, sp_n).wait()
                pltpu.make_async_copy(w_hbm.at[cb], wr_n, sp_n).wait()
                _unpack(gi_n, di_n)
                pltpu.async_copy(y_hbm.at[gi_n], rows_n, sg_n)

            pltpu.make_async_copy(y_hbm.at[gi_b], rows_b, sg_b).wait()

            def _scale(k, c2):
                wp16 = wr_b[pl.ds(k * 16, 16)]
                we = plsc.bitcast(lax.shift_left(wp16, 16), jnp.float32)
                wo = plsc.bitcast(lax.bitwise_and(wp16, m16), jnp.float32)
                for l in range(16):
                    for (e, wl16) in ((k * 32 + l, we), (k * 32 + 16 + l, wo)):
                        wl = jnp.full((16,), wl16[l], jnp.float32)
                        for c in range(DW // 16):
                            wrd = rows_b[e, pl.ds(c * 16, 16)]
                            lo = plsc.bitcast(lax.shift_left(wrd, 16),
                                              jnp.float32)
                            hi = plsc.bitcast(lax.bitwise_and(wrd, m16),
                                              jnp.float32)
                            frows_b[e, pl.ds(c * 32, 16)] = lo * wl
                            frows_b[e, pl.ds(c * 32 + 16, 16)] = hi * wl
                return c2
            lax.fori_loop(0, CH // 32, _scale, 0)
            pltpu.async_copy(frows_b, a_sh.at[di_b], ss_b, add=True)

            @pl.when(j + 2 < ncht)
            def _():
                pltpu.async_copy(pk_hbm.at[cb + j + 2], gi_b, sp_b)
                pltpu.async_copy(w_hbm.at[cb + j + 2], wr_b, sp_b)
        return carry
    lax.fori_loop(0, lax.div(ncht, 2), _pair, 0)
    pltpu.make_async_copy(frows0, a_sh.at[di0], ss0).wait()
    pltpu.make_async_copy(frows1, a_sh.at[di1], ss1).wait()
    plsc.subcore_barrier()

    def _out(t, carry):
        off = sid * 640 + t * ZCH
        pltpu.sync_copy(a_sh.at[pl.ds(off, ZCH)],
                        p_hbm.at[cid, pl.ds(off, ZCH)])
        return carry
    lax.fori_loop(0, nstr, _out, 0)


def _pack_y(h, wlo_r, whi_r):
    ylo = jnp.dot(h, wlo_r, preferred_element_type=jnp.float32
                  ).astype(jnp.bfloat16)
    yhi = jnp.dot(h, whi_r, preferred_element_type=jnp.float32
                  ).astype(jnp.bfloat16)
    lo_i = lax.bitcast_convert_type(ylo, jnp.uint16).astype(jnp.int32)
    hi_i = lax.bitcast_convert_type(yhi, jnp.uint16).astype(jnp.int32)
    return lo_i | lax.shift_left(hi_i, 16)


def _mm1_body(x_ref, root_ref, wlo_ref, whi_ref, b_ref, base_ref, y_ref):
    xb = x_ref[...]
    base_ref[...] = jnp.dot(xb, root_ref[...],
                            preferred_element_type=jnp.float32) + b_ref[...]
    for r in range(R):
        y_ref[r] = _pack_y(xb, wlo_ref[r], whi_ref[r])


def _mm1(x, root, wlo, whi, b):
    return pl.pallas_call(
        _mm1_body,
        grid=(NB,),
        in_specs=[pl.BlockSpec((BN, D), lambda i: (i, 0)),
                  pl.BlockSpec((D, D), lambda i: (0, 0)),
                  pl.BlockSpec((R, D, DW), lambda i: (0, 0, 0)),
                  pl.BlockSpec((R, D, DW), lambda i: (0, 0, 0)),
                  pl.BlockSpec((1, D), lambda i: (0, 0))],
        out_specs=[pl.BlockSpec((BN, D), lambda i: (i, 0)),
                   pl.BlockSpec((R, BN, DW), lambda i: (0, i, 0))],
        out_shape=[jax.ShapeDtypeStruct((N, D), jnp.float32),
                   jax.ShapeDtypeStruct((R, N, DW), jnp.int32)],
    )(x, root, wlo, whi, b)


def _mm2_body(base_ref, p_ref, root_ref, wlo_ref, whi_ref, b_ref,
              base2_ref, y_ref):
    h = jnp.maximum(base_ref[...] + p_ref[0] + p_ref[1], 0.0)
    base2_ref[...] = jnp.dot(h, root_ref[...],
                             preferred_element_type=jnp.float32) + b_ref[...]
    for r in range(R):
        y_ref[r] = _pack_y(h, wlo_ref[r], whi_ref[r])


def _mm2(base, p, root, wlo, whi, b):
    return pl.pallas_call(
        _mm2_body,
        grid=(NB,),
        in_specs=[pl.BlockSpec((BN, D), lambda i: (i, 0)),
                  pl.BlockSpec((NC, BN, D), lambda i: (0, i, 0)),
                  pl.BlockSpec((D, D), lambda i: (0, 0)),
                  pl.BlockSpec((R, D, DW), lambda i: (0, 0, 0)),
                  pl.BlockSpec((R, D, DW), lambda i: (0, 0, 0)),
                  pl.BlockSpec((1, D), lambda i: (0, 0))],
        out_specs=[pl.BlockSpec((BN, D), lambda i: (i, 0)),
                   pl.BlockSpec((R, BN, DW), lambda i: (0, i, 0))],
        out_shape=[jax.ShapeDtypeStruct((N, D), jnp.float32),
                   jax.ShapeDtypeStruct((R, N, DW), jnp.int32)],
    )(base, p, root, wlo, whi, b)


def _pool_body(base_ref, p_ref, batch_ref, linw_ref, linb_ref, out_ref,
               sums, cnts):
    i = pl.program_id(0)

    @pl.when(i == 0)
    def _():
        sums[...] = jnp.zeros((G, D), jnp.float32)
        cnts[...] = jnp.zeros((G, D), jnp.float32)

    h = jnp.maximum(base_ref[...] + p_ref[0] + p_ref[1], 0.0)
    b = batch_ref[...]
    oh = (b == lax.broadcasted_iota(jnp.int32, (BN, G), 1)).astype(jnp.float32)
    sums[...] += lax.dot_general(oh, h, (((0,), (0,)), ((), ())),
                                 preferred_element_type=jnp.float32)
    cnts[...] += jnp.sum(oh, axis=0)[:, None]

    @pl.when(i == NB - 1)
    def _():
        pooled = sums[...] / jnp.maximum(cnts[...], 1.0)
        out_ref[...] = jnp.dot(pooled, linw_ref[...],
                               preferred_element_type=jnp.float32) + linb_ref[...]


def _pool(base, p, batch, linw, linb):
    return pl.pallas_call(
        _pool_body,
        grid=(NB,),
        in_specs=[pl.BlockSpec((BN, D), lambda i: (i, 0)),
                  pl.BlockSpec((NC, BN, D), lambda i: (0, i, 0)),
                  pl.BlockSpec((BN, 1), lambda i: (i, 0)),
                  pl.BlockSpec((D, D), lambda i: (0, 0)),
                  pl.BlockSpec((1, D), lambda i: (0, 0))],
        out_specs=pl.BlockSpec((G, D), lambda i: (0, 0)),
        out_shape=jax.ShapeDtypeStruct((G, D), jnp.float32),
        scratch_shapes=[pltpu.VMEM((G, D), jnp.float32),
                        pltpu.VMEM((G, D), jnp.float32)],
    )(base, p, batch, linw, linb)


def kernel(x, edge_index, edge_type, batch, W1, root1, b1, W2, root2, b2,
           linW, linb):
    src = edge_index[0].astype(jnp.int32)
    dst = edge_index[1].astype(jnp.int32)
    et = edge_type.astype(jnp.int32)
    pad = E_PAD - E
    src_p = jnp.pad(src, (0, pad))
    dst_p = jnp.pad(dst, (0, pad))
    et_p = jnp.pad(et, (0, pad))

    pk2, wpk2 = _count_weights(et_p, src_p, dst_p)

    lo_sel = jnp.array(LO_SEL, dtype=jnp.int32)
    hi_sel = jnp.array(HI_SEL, dtype=jnp.int32)
    w1lo = W1[:, :, lo_sel]
    w1hi = W1[:, :, hi_sel]
    w2lo = W2[:, :, lo_sel]
    w2hi = W2[:, :, hi_sel]

    base1, y1 = _mm1(x, root1, w1lo, w1hi, b1.reshape(1, D))
    p1 = _scatter(y1.reshape(R * N, DW), pk2, wpk2)
    base2, y2 = _mm2(base1, p1, root2, w2lo, w2hi, b2.reshape(1, D))
    p2 = _scatter(y2.reshape(R * N, DW), pk2, wpk2)

    linWp = jnp.zeros((D, D), jnp.float32).at[:, :2].set(linW)
    linbp = jnp.zeros((1, D), jnp.float32).at[0, :2].set(linb)
    out = _pool(base2, p2, batch.astype(jnp.int32).reshape(N, 1),
                linWp, linbp)
    return out[:, :2]


# speedup vs baseline: 1.7457x; 1.1732x over previous
import functools

import jax
import jax.numpy as jnp
from jax import lax
from jax.experimental import pallas as pl
from jax.experimental.pallas import tpu as pltpu
from jax.experimental.pallas import tpu_sc as plsc

N = 10000
E = 320000
D = 128
DW = D // 2
R = 4
G = 8
NC = 2
NS = 16
NW = NC * NS

E_PAD = 327680
NCHG = E_PAD // 128
EPT = E_PAD // NW
EPC = E_PAD // NS
CH = 128
WCH = 2048
RNP = 40960
SL = RNP // NS
N_PAD = 10112
ZCH = 128
BN = 400
NB = N // BN
_DBITS = 14
NCH0 = 80
NCH1 = 80

LO_SEL = [32 * (c // 16) + (c % 16) for c in range(DW)]
HI_SEL = [32 * (c // 16) + 16 + (c % 16) for c in range(DW)]

_sc_mesh = plsc.VectorSubcoreMesh(
    core_axis_name="c", subcore_axis_name="s", num_cores=NC, num_subcores=NS)


@functools.partial(
    pl.kernel,
    out_type=[jax.ShapeDtypeStruct((NCHG, CH), jnp.int32),
              jax.ShapeDtypeStruct((NCHG, CH // 2), jnp.int32)],
    mesh=_sc_mesh,
    scratch_types=[
        pltpu.VMEM_SHARED((RNP,), jnp.float32),
        pltpu.VMEM((RNP,), jnp.float32),
        pltpu.VMEM((SL,), jnp.float32),
        pltpu.VMEM((WCH // CH, CH), jnp.int32),
        pltpu.VMEM((WCH // CH, CH), jnp.float32),
        pltpu.SemaphoreType.DMA,
        pltpu.VMEM((WCH,), jnp.int32),
        pltpu.VMEM((WCH,), jnp.int32),
        pltpu.VMEM((WCH,), jnp.int32),
        pltpu.VMEM((WCH // CH, CH), jnp.int32),
        pltpu.VMEM((WCH // CH, CH // 2), jnp.int32),
    ],
    compiler_params=pltpu.CompilerParams(needs_layout_passes=False),
)
def _count_weights(et_hbm, src_hbm, dst_hbm, pk_hbm, wpk_hbm,
                   c_sh, cl, zb, sidx2, val2, csem,
                   etb2, srcb2, dstb2, gb2, wb2):
    cid = lax.axis_index("c")
    sid = lax.axis_index("s")
    zero16 = jnp.zeros((16,), jnp.float32)
    iota16 = lax.broadcasted_iota(jnp.int32, (16,), 0)

    def _z(i, carry):
        zb[pl.ds(i * 16, 16)] = zero16
        return carry
    lax.fori_loop(0, SL // 16, _z, 0)
    pltpu.sync_copy(zb, c_sh.at[pl.ds(sid * SL, SL)])
    plsc.subcore_barrier()

    def _cblock(jb, carry):
        base = sid * EPC + jb * WCH
        pltpu.sync_copy(et_hbm.at[pl.ds(base, WCH)], etb2)
        pltpu.sync_copy(dst_hbm.at[pl.ds(base, WCH)], dstb2)

        def _grp(k, c2):
            o = k * 16
            s16 = etb2[pl.ds(o, 16)] * N + dstb2[pl.ds(o, 16)]
            v16 = jnp.where(base + o + iota16 < E, 1.0, 0.0)
            sidx2[k // 8, pl.ds((k % 8) * 16, 16)] = s16
            val2[k // 8, pl.ds((k % 8) * 16, 16)] = v16
            return c2
        lax.fori_loop(0, WCH // 16, _grp, 0)

        def _fire(k, c2):
            pltpu.async_copy(val2.at[k], c_sh.at[sidx2.at[k]], csem, add=True)
            return c2
        lax.fori_loop(0, WCH // CH, _fire, 0)

        def _drain(k, c2):
            pltpu.make_async_copy(val2.at[0], c_sh.at[sidx2.at[0]],
                                  csem).wait()
            return c2
        lax.fori_loop(0, WCH // CH, _drain, 0)
        return carry
    lax.fori_loop(0, EPC // WCH, _cblock, 0)
    plsc.subcore_barrier()
    pltpu.sync_copy(c_sh, cl)

    wid = cid * NS + sid

    def _w16(base, o):
        et16 = etb2[pl.ds(o, 16)]
        d16 = dstb2[pl.ds(o, 16)]
        g16 = et16 * N + srcb2[pl.ds(o, 16)]
        c16 = plsc.load_gather(cl, [et16 * N + d16])
        w16 = jnp.where(base + o + iota16 < E,
                        1.0 / jnp.maximum(c16, 1.0), 0.0)
        return lax.shift_left(g16, _DBITS) | d16, w16

    def _wchunk(j, carry):
        base = wid * EPT + j * WCH
        pltpu.sync_copy(et_hbm.at[pl.ds(base, WCH)], etb2)
        pltpu.sync_copy(src_hbm.at[pl.ds(base, WCH)], srcb2)
        pltpu.sync_copy(dst_hbm.at[pl.ds(base, WCH)], dstb2)

        def _grp(k, c2):
            pk16a, wa = _w16(base, k * 32)
            pk16b, wb = _w16(base, k * 32 + 16)
            row = k // 4
            col = (k % 4) * 32
            gb2[row, pl.ds(col, 16)] = pk16a
            gb2[row, pl.ds(col + 16, 16)] = pk16b
            wword = lax.bitwise_or(
                lax.shift_right_logical(plsc.bitcast(wa, jnp.int32), 16),
                lax.bitwise_and(plsc.bitcast(wb, jnp.int32),
                                jnp.int32(-65536)))
            wb2[row, pl.ds((k % 4) * 16, 16)] = wword
            return c2
        lax.fori_loop(0, WCH // 32, _grp, 0)
        rows0 = wid * (EPT // CH) + j * (WCH // CH)
        pltpu.sync_copy(gb2, pk_hbm.at[pl.ds(rows0, WCH // CH)])
        pltpu.sync_copy(wb2, wpk_hbm.at[pl.ds(rows0, WCH // CH)])
        return carry
    lax.fori_loop(0, EPT // WCH, _wchunk, 0)


@functools.partial(
    pl.kernel,
    out_type=jax.ShapeDtypeStruct((NC, N_PAD, D), jnp.float32),
    mesh=_sc_mesh,
    scratch_types=[
        pltpu.VMEM_SHARED((N_PAD, D), jnp.float32),
        pltpu.VMEM((CH // 2,), jnp.int32),
        pltpu.VMEM((CH // 2,), jnp.int32),
        pltpu.VMEM((CH,), jnp.int32),
        pltpu.VMEM((CH,), jnp.int32),
        pltpu.VMEM((CH,), jnp.int32),
        pltpu.VMEM((CH,), jnp.int32),
        pltpu.VMEM((CH, DW), jnp.int32),
        pltpu.VMEM((CH, DW), jnp.int32),
        pltpu.VMEM((CH, D), jnp.float32),
        pltpu.VMEM((CH, D), jnp.float32),
        pltpu.SemaphoreType.DMA,
        pltpu.SemaphoreType.DMA,
        pltpu.SemaphoreType.DMA,
        pltpu.SemaphoreType.DMA,
        pltpu.SemaphoreType.DMA,
        pltpu.SemaphoreType.DMA,
    ],
    compiler_params=pltpu.CompilerParams(needs_layout_passes=False,
                                          use_tc_tiling_on_sc=False),
)
def _scatter(y_hbm, pk_hbm, w_hbm, p_hbm,
             a_sh, wr0, wr1, gi0, gi1, di0, di1,
             rows0, rows1, frows0, frows1, sp0, sp1, sg0, sg1, ss0, ss1):
    cid = lax.axis_index("c")
    sid = lax.axis_index("s")
    zero16 = jnp.zeros((16,), jnp.float32)
    m16 = jnp.int32(-65536)

    def _z(i, carry):
        frows0[i // 8, pl.ds((i % 8) * 16, 16)] = zero16
        return carry
    lax.fori_loop(0, (CH * D) // 16, _z, 0)

    nstr = jnp.where(sid == NS - 1, 4, 5)

    def _zs(t, carry):
        pltpu.sync_copy(frows0, a_sh.at[pl.ds(sid * 640 + t * ZCH, ZCH)])
        return carry
    lax.fori_loop(0, nstr, _zs, 0)
    plsc.subcore_barrier()

    ncht = jnp.where(cid == 0, NCH0, NCH1)
    cb = jnp.where(cid == 0, sid * NCH0, NS * NCH0 + sid * NCH1)

    def _unpack(gi_s, di_s):
        for k in range(CH // 16):
            o = k * 16
            p16 = gi_s[pl.ds(o, 16)]
            di_s[pl.ds(o, 16)] = lax.bitwise_and(p16, (1 << _DBITS) - 1)
            gi_s[pl.ds(o, 16)] = lax.shift_right_logical(p16, _DBITS)

    bufs = ((rows0, frows0, wr0, gi0, di0, sp0, sg0, ss0),
            (rows1, frows1, wr1, gi1, di1, sp1, sg1, ss1))

    pltpu.sync_copy(pk_hbm.at[cb], gi0)
    pltpu.sync_copy(w_hbm.at[cb], wr0)
    _unpack(gi0, di0)
    pltpu.async_copy(y_hbm.at[gi0], rows0, sg0)
    pltpu.async_copy(pk_hbm.at[cb + 1], gi1, sp1)
    pltpu.async_copy(w_hbm.at[cb + 1], wr1, sp1)

    def _pair(jj, carry):
        for b in range(2):
            j = jj * 2 + b
            rows_b, frows_b, wr_b, gi_b, di_b, sp_b, sg_b, ss_b = bufs[b]
            rows_n, frows_n, wr_n, gi_n, di_n, sp_n, sg_n, ss_n = bufs[1 - b]

            @pl.when(j + 1 < ncht)
            def _():
                @pl.when(j >= 1)
                def _():
                    pltpu.make_async_copy(frows_n, a_sh.at[di_n],
                                          ss_n).wait()
                pltpu.make_async_copy(pk_hbm.at[cb], gi_n, sp_n).wait()
                pltpu.make_async_copy(w_hbm.at[cb], wr_n, sp_n).wait()
                _unpack(gi_n, di_n)
                pltpu.async_copy(y_hbm.at[gi_n], rows_n, sg_n)

            pltpu.make_async_copy(y_hbm.at[gi_b], rows_b, sg_b).wait()

            def _scale(k, c2):
                wp16 = wr_b[pl.ds(k * 16, 16)]
                we = plsc.bitcast(lax.shift_left(wp16, 16), jnp.float32)
                wo = plsc.bitcast(lax.bitwise_and(wp16, m16), jnp.float32)
                for l in range(16):
                    for (e, wl16) in ((k * 32 + l, we), (k * 32 + 16 + l, wo)):
                        wl = jnp.full((16,), wl16[l], jnp.float32)
                        for c in range(DW // 16):
                            wrd = rows_b[e, pl.ds(c * 16, 16)]
                            lo = plsc.bitcast(lax.shift_left(wrd, 16),
                                              jnp.float32)
                            hi = plsc.bitcast(lax.bitwise_and(wrd, m16),
                                              jnp.float32)
                            frows_b[e, pl.ds(c * 32, 16)] = lo * wl
                            frows_b[e, pl.ds(c * 32 + 16, 16)] = hi * wl
                return c2
            lax.fori_loop(0, CH // 32, _scale, 0)
            pltpu.async_copy(frows_b, a_sh.at[di_b], ss_b, add=True)

            @pl.when(j + 2 < ncht)
            def _():
                pltpu.async_copy(pk_hbm.at[cb + j + 2], gi_b, sp_b)
                pltpu.async_copy(w_hbm.at[cb + j + 2], wr_b, sp_b)
        return carry
    lax.fori_loop(0, lax.div(ncht, 2), _pair, 0)
    pltpu.make_async_copy(frows0, a_sh.at[di0], ss0).wait()
    pltpu.make_async_copy(frows1, a_sh.at[di1], ss1).wait()
    plsc.subcore_barrier()

    def _out(t, carry):
        off = sid * 640 + t * ZCH
        pltpu.sync_copy(a_sh.at[pl.ds(off, ZCH)],
                        p_hbm.at[cid, pl.ds(off, ZCH)])
        return carry
    lax.fori_loop(0, nstr, _out, 0)


def _pack_y(h, wlo_r, whi_r):
    ylo = jnp.dot(h, wlo_r, preferred_element_type=jnp.float32
                  ).astype(jnp.bfloat16)
    yhi = jnp.dot(h, whi_r, preferred_element_type=jnp.float32
                  ).astype(jnp.bfloat16)
    lo_i = lax.bitcast_convert_type(ylo, jnp.uint16).astype(jnp.int32)
    hi_i = lax.bitcast_convert_type(yhi, jnp.uint16).astype(jnp.int32)
    return lo_i | lax.shift_left(hi_i, 16)


def _mm1_body(x_ref, root_ref, wlo_ref, whi_ref, b_ref, base_ref, y_ref):
    xb = x_ref[...]
    base_ref[...] = jnp.dot(xb, root_ref[...],
                            preferred_element_type=jnp.float32) + b_ref[...]
    for r in range(R):
        y_ref[r] = _pack_y(xb, wlo_ref[r], whi_ref[r])


def _mm1(x, root, wlo, whi, b):
    return pl.pallas_call(
        _mm1_body,
        grid=(NB,),
        in_specs=[pl.BlockSpec((BN, D), lambda i: (i, 0)),
                  pl.BlockSpec((D, D), lambda i: (0, 0)),
                  pl.BlockSpec((R, D, DW), lambda i: (0, 0, 0)),
                  pl.BlockSpec((R, D, DW), lambda i: (0, 0, 0)),
                  pl.BlockSpec((1, D), lambda i: (0, 0))],
        out_specs=[pl.BlockSpec((BN, D), lambda i: (i, 0)),
                   pl.BlockSpec((R, BN, DW), lambda i: (0, i, 0))],
        out_shape=[jax.ShapeDtypeStruct((N, D), jnp.float32),
                   jax.ShapeDtypeStruct((R, N, DW), jnp.int32)],
    )(x, root, wlo, whi, b)


def _mm2_body(base_ref, p_ref, root_ref, wlo_ref, whi_ref, b_ref,
              base2_ref, y_ref):
    h = jnp.maximum(base_ref[...] + p_ref[0] + p_ref[1], 0.0)
    base2_ref[...] = jnp.dot(h, root_ref[...],
                             preferred_element_type=jnp.float32) + b_ref[...]
    for r in range(R):
        y_ref[r] = _pack_y(h, wlo_ref[r], whi_ref[r])


def _mm2(base, p, root, wlo, whi, b):
    return pl.pallas_call(
        _mm2_body,
        grid=(NB,),
        in_specs=[pl.BlockSpec((BN, D), lambda i: (i, 0)),
                  pl.BlockSpec((NC, BN, D), lambda i: (0, i, 0)),
                  pl.BlockSpec((D, D), lambda i: (0, 0)),
                  pl.BlockSpec((R, D, DW), lambda i: (0, 0, 0)),
                  pl.BlockSpec((R, D, DW), lambda i: (0, 0, 0)),
                  pl.BlockSpec((1, D), lambda i: (0, 0))],
        out_specs=[pl.BlockSpec((BN, D), lambda i: (i, 0)),
                   pl.BlockSpec((R, BN, DW), lambda i: (0, i, 0))],
        out_shape=[jax.ShapeDtypeStruct((N, D), jnp.float32),
                   jax.ShapeDtypeStruct((R, N, DW), jnp.int32)],
    )(base, p, root, wlo, whi, b)


def _pool_body(base_ref, p_ref, batch_ref, linw_ref, linb_ref, out_ref,
               sums, cnts):
    i = pl.program_id(0)

    @pl.when(i == 0)
    def _():
        sums[...] = jnp.zeros((G, D), jnp.float32)
        cnts[...] = jnp.zeros((G, D), jnp.float32)

    h = jnp.maximum(base_ref[...] + p_ref[0] + p_ref[1], 0.0)
    b = batch_ref[...]
    oh = (b == lax.broadcasted_iota(jnp.int32, (BN, G), 1)).astype(jnp.float32)
    sums[...] += lax.dot_general(oh, h, (((0,), (0,)), ((), ())),
                                 preferred_element_type=jnp.float32)
    cnts[...] += jnp.sum(oh, axis=0)[:, None]

    @pl.when(i == NB - 1)
    def _():
        pooled = sums[...] / jnp.maximum(cnts[...], 1.0)
        out_ref[...] = jnp.dot(pooled, linw_ref[...],
                               preferred_element_type=jnp.float32) + linb_ref[...]


def _pool(base, p, batch, linw, linb):
    return pl.pallas_call(
        _pool_body,
        grid=(NB,),
        in_specs=[pl.BlockSpec((BN, D), lambda i: (i, 0)),
                  pl.BlockSpec((NC, BN, D), lambda i: (0, i, 0)),
                  pl.BlockSpec((BN, 1), lambda i: (i, 0)),
                  pl.BlockSpec((D, D), lambda i: (0, 0)),
                  pl.BlockSpec((1, D), lambda i: (0, 0))],
        out_specs=pl.BlockSpec((G, D), lambda i: (0, 0)),
        out_shape=jax.ShapeDtypeStruct((G, D), jnp.float32),
        scratch_shapes=[pltpu.VMEM((G, D), jnp.float32),
                        pltpu.VMEM((G, D), jnp.float32)],
    )(base, p, batch, linw, linb)


def kernel(x, edge_index, edge_type, batch, W1, root1, b1, W2, root2, b2,
           linW, linb):
    src = edge_index[0].astype(jnp.int32)
    dst = edge_index[1].astype(jnp.int32)
    et = edge_type.astype(jnp.int32)
    pad = E_PAD - E
    src_p = jnp.pad(src, (0, pad))
    dst_p = jnp.pad(dst, (0, pad))
    et_p = jnp.pad(et, (0, pad))

    pk2, wpk2 = _count_weights(et_p, src_p, dst_p)

    lo_sel = jnp.array(LO_SEL, dtype=jnp.int32)
    hi_sel = jnp.array(HI_SEL, dtype=jnp.int32)
    w1lo = W1[:, :, lo_sel]
    w1hi = W1[:, :, hi_sel]
    w2lo = W2[:, :, lo_sel]
    w2hi = W2[:, :, hi_sel]

    base1, y1 = _mm1(x, root1, w1lo, w1hi, b1.reshape(1, D))
    p1 = _scatter(y1.reshape(R * N, DW), pk2, wpk2)
    base2, y2 = _mm2(base1, p1, root2, w2lo, w2hi, b2.reshape(1, D))
    p2 = _scatter(y2.reshape(R * N, DW), pk2, wpk2)

    linWp = jnp.zeros((D, D), jnp.float32).at[:, :2].set(linW)
    linbp = jnp.zeros((1, D), jnp.float32).at[0, :2].set(linb)
    out = _pool(base2, p2, batch.astype(jnp.int32).reshape(N, 1),
                linWp, linbp)
    return out[:, :2]
